# baseline (speedup 1.0000x reference)
import functools
import jax
import jax.numpy as jnp
from jax import lax
from jax.experimental import pallas as pl
from jax.experimental.pallas import tpu as pltpu
from jax.experimental.pallas import tpu_sc as plsc

M_CENTROIDS = 512
RADIUS = 0.15
KNBR = 32
EPS_BN = 1e-5
_NB = 8
_NN = 2048
_L = 16
_NW = 32
_WQ = _NW // _NB
_MW = M_CENTROIDS // _WQ
_GCHUNK = 256


def _prep_body(ptsT_ref, featT_ref, w1xT_ref, w1fT_ref, h_ref):
    ptsT = ptsT_ref[0]
    featT = featT_ref[0]
    h = jnp.dot(featT, w1fT_ref[...], preferred_element_type=jnp.float32)
    h = h + jnp.dot(ptsT, w1xT_ref[...], preferred_element_type=jnp.float32)
    h_ref[0] = h


def _prep_h(ptsT, featT, w1xT, w1fT):
    B, N, _ = ptsT.shape
    return pl.pallas_call(
        _prep_body,
        grid=(B,),
        in_specs=[
            pl.BlockSpec((1, N, 3), lambda b: (b, 0, 0)),
            pl.BlockSpec((1, N, featT.shape[2]), lambda b: (b, 0, 0)),
            pl.BlockSpec(w1xT.shape, lambda b: (0, 0)),
            pl.BlockSpec(w1fT.shape, lambda b: (0, 0)),
        ],
        out_specs=pl.BlockSpec((1, N, 128), lambda b: (b, 0, 0)),
        out_shape=jax.ShapeDtypeStruct((B, N, 128), jnp.float32),
    )(ptsT, featT, w1xT, w1fT)


def _mlp_body(g_ref, nxT_ref, w1xT_ref, w2T_ref, w3T_ref,
              s1_ref, b1_ref, s2_ref, b2_ref, s3_ref, b3_ref, out_ref):
    Mt = g_ref.shape[1]
    g = g_ref[0]
    nx = nxT_ref[0]
    cc = jnp.dot(nx, w1xT_ref[...], preferred_element_type=jnp.float32)
    y1 = g - cc[:, None, :]
    y1 = jnp.maximum(y1 * s1_ref[0][None, None, :] + b1_ref[0][None, None, :], 0.0)
    x1 = y1.reshape(Mt * KNBR, 128)
    y2 = jnp.dot(x1, w2T_ref[...], preferred_element_type=jnp.float32)
    y2 = jnp.maximum(y2 * s2_ref[0][None, :] + b2_ref[0][None, :], 0.0)
    y3 = jnp.dot(y2, w3T_ref[...], preferred_element_type=jnp.float32)
    y3 = jnp.maximum(y3 * s3_ref[0][None, :] + b3_ref[0][None, :], 0.0)
    m3 = y3.reshape(Mt, KNBR, 256)
    r = m3[:, 0, :]
    for k in range(1, KNBR):
        r = jnp.maximum(r, m3[:, k, :])
    out_ref[0] = r


def _mlp_maxpool(G, new_ptsT, w1xT, w2T, w3T, s1, b1, s2, b2, s3, b3):
    B, M, K, _ = G.shape
    Mt = 128
    vec = lambda v: v.reshape(1, -1)
    return pl.pallas_call(
        _mlp_body,
        grid=(B, M // Mt),
        in_specs=[
            pl.BlockSpec((1, Mt, K, 128), lambda b, m: (b, m, 0, 0)),
            pl.BlockSpec((1, Mt, 3), lambda b, m: (b, m, 0)),
            pl.BlockSpec((3, 128), lambda b, m: (0, 0)),
            pl.BlockSpec((128, 128), lambda b, m: (0, 0)),
            pl.BlockSpec((128, 256), lambda b, m: (0, 0)),
            pl.BlockSpec((1, 128), lambda b, m: (0, 0)),
            pl.BlockSpec((1, 128), lambda b, m: (0, 0)),
            pl.BlockSpec((1, 128), lambda b, m: (0, 0)),
            pl.BlockSpec((1, 128), lambda b, m: (0, 0)),
            pl.BlockSpec((1, 256), lambda b, m: (0, 0)),
            pl.BlockSpec((1, 256), lambda b, m: (0, 0)),
        ],
        out_specs=pl.BlockSpec((1, Mt, 256), lambda b, m: (b, m, 0)),
        out_shape=jax.ShapeDtypeStruct((B, M, 256), jnp.float32),
    )(G, new_ptsT, w1xT, w2T, w3T,
      vec(s1), vec(b1), vec(s2), vec(b2), vec(s3), vec(b3))


def _sc_sparse_body(xyz_hbm, h_hbm, npts_hbm, g_hbm,
                    x_v, y_v, z_v, mind_v, fps_v, bidx_v, npts_v, rows_v, sem):
    cix = lax.axis_index("c")
    six = lax.axis_index("s")
    wid = six * 2 + cix
    b = wid // _WQ
    q = wid % _WQ

    iota = lax.iota(jnp.int32, _L)
    zeros16 = jnp.full((_L,), b * 0, jnp.int32)
    lane0 = iota == 0

    pltpu.sync_copy(xyz_hbm.at[pl.ds(b * 3 * _NN, _NN)], x_v)
    pltpu.sync_copy(xyz_hbm.at[pl.ds(b * 3 * _NN + _NN, _NN)], y_v)
    pltpu.sync_copy(xyz_hbm.at[pl.ds(b * 3 * _NN + 2 * _NN, _NN)], z_v)

    big = jnp.full((_L,), 1e10, jnp.float32)

    def init_body(j, carry):
        mind_v[pl.ds(j * _L, _L)] = big
        return carry
    lax.fori_loop(0, _NN // _L, init_body, 0)

    def initf_body(j, carry):
        fps_v[pl.ds(j * _L, _L)] = iota * 0
        return carry
    lax.fori_loop(0, M_CENTROIDS // _L, initf_body, 0)
    def initb_body(j, carry):
        bidx_v[pl.ds(j * _L, _L)] = iota * 0 + b * _NN
        return carry
    lax.fori_loop(0, _MW * KNBR // _L, initb_body, 0)
    xv0 = x_v[pl.ds(0, _L)]
    yv0 = y_v[pl.ds(0, _L)]
    zv0 = z_v[pl.ds(0, _L)]
    nbig = jnp.float32(-3.4e38)
    lastx = jnp.full((_L,), jnp.max(jnp.where(lane0, xv0, nbig)), jnp.float32)
    lasty = jnp.full((_L,), jnp.max(jnp.where(lane0, yv0, nbig)), jnp.float32)
    lastz = jnp.full((_L,), jnp.max(jnp.where(lane0, zv0, nbig)), jnp.float32)
    neginf = jnp.full((_L,), -3.4e38, jnp.float32)

    def fps_step(i, carry):
        lx, ly, lz = carry

        def sweep(jj, c2):
            bestd, besti = c2
            for u in range(4):
                off = jj * (4 * _L) + u * _L
                xv = x_v[pl.ds(off, _L)]
                yv = y_v[pl.ds(off, _L)]
                zv = z_v[pl.ds(off, _L)]
                dx = xv - lx
                dy = yv - ly
                dz = zv - lz
                d2 = (dx * dx + dy * dy) + dz * dz
                md = jnp.minimum(mind_v[pl.ds(off, _L)], d2)
                mind_v[pl.ds(off, _L)] = md
                upd = md > bestd
                bestd = jnp.where(upd, md, bestd)
                besti = jnp.where(upd, iota + off, besti)
            return bestd, besti

        bestd, besti = lax.fori_loop(0, _NN // (4 * _L), sweep,
                                     (neginf, zeros16))
        mx = jnp.max(bestd)
        cand = jnp.where(bestd == mx, besti, _NN)
        nxt = jnp.min(cand)
        nxtv = jnp.full((_L,), nxt, jnp.int32)
        plsc.store_scatter(fps_v, [jnp.full((_L,), i, jnp.int32)], nxtv,
                           mask=lane0)
        return (plsc.load_gather(x_v, [nxtv]),
                plsc.load_gather(y_v, [nxtv]),
                plsc.load_gather(z_v, [nxtv]))

    lax.fori_loop(1, M_CENTROIDS, fps_step, (lastx, lasty, lastz))

    for g in range(_MW // _L):
        cidxv = fps_v[pl.ds(q * _MW + g * _L, _L)]
        cx = plsc.load_gather(x_v, [cidxv])
        cy = plsc.load_gather(y_v, [cidxv])
        cz = plsc.load_gather(z_v, [cidxv])
        rowbase = (g * _L + iota) * 3
        plsc.store_scatter(npts_v, [rowbase], cx)
        plsc.store_scatter(npts_v, [rowbase + 1], cy)
        plsc.store_scatter(npts_v, [rowbase + 2], cz)
    pltpu.sync_copy(
        npts_v,
        npts_hbm.at[pl.ds(b * M_CENTROIDS * 3 + q * _MW * 3, _MW * 3)])

    r2 = jnp.float32(RADIUS * RADIUS)

    def ball_one(cm, carry):
        pos_all = q * _MW + cm
        vbase = (pos_all // _L) * _L
        lane = pos_all % _L
        grp = fps_v[pl.ds(vbase, _L)]
        cs = jnp.max(jnp.where(iota == lane, grp, -1))
        csv = jnp.full((_L,), cs, jnp.int32)
        cx = plsc.load_gather(x_v, [csv])
        cy = plsc.load_gather(y_v, [csv])
        cz = plsc.load_gather(z_v, [csv])
        base_b = cm * KNBR

        def scan(j, cnt):
            off = j * _L
            dx = x_v[pl.ds(off, _L)] - cx
            dy = y_v[pl.ds(off, _L)] - cy
            dz = z_v[pl.ds(off, _L)] - cz
            d2 = (dx * dx + dy * dy) + dz * dz
            msk = d2 < r2
            cums = plsc.cumsum(msk.astype(jnp.int32))
            pos = cnt + cums - 1
            wm = msk & (pos < KNBR)
            plsc.store_scatter(bidx_v, [base_b + pos],
                               iota + (off + b * _NN), mask=wm)
            pc = plsc.all_reduce_population_count(msk)
            return cnt + pc

        cnt = lax.fori_loop(0, _NN // _L, scan, zeros16)

        firstv = plsc.load_gather(bidx_v, [jnp.full((_L,), base_b, jnp.int32)])
        for u in range(KNBR // _L):
            sl = pl.ds(base_b + u * _L, _L)
            cur = bidx_v[sl]
            lanepos = iota + u * _L
            bidx_v[sl] = jnp.where(lanepos < cnt, cur, firstv)
        return carry

    lax.fori_loop(0, 8, ball_one, 0)

    for i in range(_MW * KNBR // _GCHUNK):
        idx_sl = bidx_v.at[pl.ds(i * _GCHUNK, _GCHUNK)]
        pltpu.async_copy(h_hbm.at[idx_sl], rows_v, sem).wait()
        row0 = (b * M_CENTROIDS + q * _MW) * KNBR + i * _GCHUNK
        pltpu.sync_copy(rows_v, g_hbm.at[pl.ds(row0, _GCHUNK), :])


def _sc_sparse(xyz, H):
    B, _, N = xyz.shape
    mesh = plsc.VectorSubcoreMesh(core_axis_name="c", subcore_axis_name="s")
    f = pl.kernel(
        _sc_sparse_body,
        mesh=mesh,
        compiler_params=pltpu.CompilerParams(needs_layout_passes=False),
        out_type=[
            jax.ShapeDtypeStruct((B * M_CENTROIDS * 3,), jnp.float32),
            jax.ShapeDtypeStruct((B * M_CENTROIDS * KNBR, 128), jnp.float32),
        ],
        scratch_types=[
            pltpu.VMEM((N,), jnp.float32),
            pltpu.VMEM((N,), jnp.float32),
            pltpu.VMEM((N,), jnp.float32),
            pltpu.VMEM((N,), jnp.float32),
            pltpu.VMEM((M_CENTROIDS,), jnp.int32),
            pltpu.VMEM((_MW * KNBR,), jnp.int32),
            pltpu.VMEM((_MW * 3,), jnp.float32),
            pltpu.VMEM((_GCHUNK, 128), jnp.float32),
            pltpu.SemaphoreType.DMA,
        ],
    )
    return f(xyz.reshape(-1), H.reshape(-1, 128))


def _gather_rows(x, idx):
    return jax.vmap(lambda xb, ib: xb[ib])(x, idx)


def _fps_scaffold(pts, M):
    B, N, _ = pts.shape
    def body(i, state):
        idxs, min_d, last = state
        last_pt = _gather_rows(pts, last)
        d = jnp.sum((pts - last_pt[:, None, :]) ** 2, axis=-1)
        min_d = jnp.minimum(min_d, d)
        nxt = jnp.argmax(min_d, axis=-1).astype(jnp.int32)
        idxs = idxs.at[:, i].set(nxt)
        return (idxs, min_d, nxt)
    state0 = (jnp.zeros((B, M), jnp.int32),
              jnp.full((B, N), 1e10, jnp.float32),
              jnp.zeros((B,), jnp.int32))
    idxs, _, _ = lax.fori_loop(1, M, body, state0)
    return idxs


def _ball_scaffold(new_pts, pts, radius, K):
    d2 = jnp.sum((new_pts[:, :, None, :] - pts[:, None, :, :]) ** 2, axis=-1)
    N = pts.shape[1]
    cand = jnp.where(d2 < radius * radius,
                     jnp.arange(N, dtype=jnp.int32)[None, None, :], N)
    idx_sorted = jnp.sort(cand, axis=-1)[:, :, :K]
    first = idx_sorted[:, :, 0:1]
    idx = jnp.where(idx_sorted < N, idx_sorted, first)
    return jnp.minimum(idx, N - 1).astype(jnp.int32)


def kernel(xyz, feature, W1, gamma1, beta1, W2, gamma2, beta2, W3, gamma3, beta3):
    B, _, N = xyz.shape
    inv = 1.0 / jnp.sqrt(1.0 + EPS_BN)
    s1, s2, s3 = gamma1 * inv, gamma2 * inv, gamma3 * inv

    ptsT = jnp.transpose(xyz, (0, 2, 1))
    featT = jnp.transpose(feature, (0, 2, 1))
    w1xT = jnp.transpose(W1[:, :3])
    w1fT = jnp.transpose(W1[:, 3:])
    w2T = jnp.transpose(W2)
    w3T = jnp.transpose(W3)

    H = _prep_h(ptsT, featT, w1xT, w1fT)

    npts_flat, G_flat = _sc_sparse(xyz, H)
    new_pts = npts_flat.reshape(B, M_CENTROIDS, 3)
    G = G_flat.reshape(B, M_CENTROIDS, KNBR, 128)

    nf = _mlp_maxpool(G, new_pts, w1xT, w2T, w3T, s1, beta1, s2, beta2, s3, beta3)
    new_xyz = jnp.transpose(new_pts, (0, 2, 1))
    new_feature = jnp.transpose(nf, (0, 2, 1))
    return (new_xyz, new_feature)


# speedup vs baseline: 1.8400x; 1.8400x over previous
import functools
import jax
import jax.numpy as jnp
from jax import lax
from jax.experimental import pallas as pl
from jax.experimental.pallas import tpu as pltpu
from jax.experimental.pallas import tpu_sc as plsc

M_CENTROIDS = 512
RADIUS = 0.15
KNBR = 32
EPS_BN = 1e-5
_NB = 8
_NN = 2048
_L = 16
_NW = 32
_WQ = _NW // _NB
_MW = M_CENTROIDS // _WQ
_GCHUNK = 256


def _prep_body(ptsT_ref, featT_ref, w1xT_ref, w1fT_ref, h_ref):
    ptsT = ptsT_ref[0]
    featT = featT_ref[0]
    h = jnp.dot(featT, w1fT_ref[...], preferred_element_type=jnp.float32)
    h = h + jnp.dot(ptsT, w1xT_ref[...], preferred_element_type=jnp.float32)
    h_ref[0] = h


def _prep_h(ptsT, featT, w1xT, w1fT):
    B, N, _ = ptsT.shape
    return pl.pallas_call(
        _prep_body,
        grid=(B,),
        in_specs=[
            pl.BlockSpec((1, N, 3), lambda b: (b, 0, 0)),
            pl.BlockSpec((1, N, featT.shape[2]), lambda b: (b, 0, 0)),
            pl.BlockSpec(w1xT.shape, lambda b: (0, 0)),
            pl.BlockSpec(w1fT.shape, lambda b: (0, 0)),
        ],
        out_specs=pl.BlockSpec((1, N, 128), lambda b: (b, 0, 0)),
        out_shape=jax.ShapeDtypeStruct((B, N, 128), jnp.float32),
    )(ptsT, featT, w1xT, w1fT)


def _mlp_body(g_ref, nxT_ref, w1xT_ref, w2T_ref, w3T_ref,
              s1_ref, b1_ref, s2_ref, b2_ref, s3_ref, b3_ref, out_ref):
    Mt = g_ref.shape[1]
    g = g_ref[0]
    nx = nxT_ref[0]
    cc = jnp.dot(nx, w1xT_ref[...], preferred_element_type=jnp.float32)
    y1 = g - cc[:, None, :]
    y1 = jnp.maximum(y1 * s1_ref[0][None, None, :] + b1_ref[0][None, None, :], 0.0)
    x1 = y1.reshape(Mt * KNBR, 128)
    y2 = jnp.dot(x1, w2T_ref[...], preferred_element_type=jnp.float32)
    y2 = jnp.maximum(y2 * s2_ref[0][None, :] + b2_ref[0][None, :], 0.0)
    y3 = jnp.dot(y2, w3T_ref[...], preferred_element_type=jnp.float32)
    y3 = jnp.maximum(y3 * s3_ref[0][None, :] + b3_ref[0][None, :], 0.0)
    m3 = y3.reshape(Mt, KNBR, 256)
    r = m3[:, 0, :]
    for k in range(1, KNBR):
        r = jnp.maximum(r, m3[:, k, :])
    out_ref[0] = r


def _mlp_maxpool(G, new_ptsT, w1xT, w2T, w3T, s1, b1, s2, b2, s3, b3):
    B, M, K, _ = G.shape
    Mt = 128
    vec = lambda v: v.reshape(1, -1)
    return pl.pallas_call(
        _mlp_body,
        grid=(B, M // Mt),
        in_specs=[
            pl.BlockSpec((1, Mt, K, 128), lambda b, m: (b, m, 0, 0)),
            pl.BlockSpec((1, Mt, 3), lambda b, m: (b, m, 0)),
            pl.BlockSpec((3, 128), lambda b, m: (0, 0)),
            pl.BlockSpec((128, 128), lambda b, m: (0, 0)),
            pl.BlockSpec((128, 256), lambda b, m: (0, 0)),
            pl.BlockSpec((1, 128), lambda b, m: (0, 0)),
            pl.BlockSpec((1, 128), lambda b, m: (0, 0)),
            pl.BlockSpec((1, 128), lambda b, m: (0, 0)),
            pl.BlockSpec((1, 128), lambda b, m: (0, 0)),
            pl.BlockSpec((1, 256), lambda b, m: (0, 0)),
            pl.BlockSpec((1, 256), lambda b, m: (0, 0)),
        ],
        out_specs=pl.BlockSpec((1, Mt, 256), lambda b, m: (b, m, 0)),
        out_shape=jax.ShapeDtypeStruct((B, M, 256), jnp.float32),
    )(G, new_ptsT, w1xT, w2T, w3T,
      vec(s1), vec(b1), vec(s2), vec(b2), vec(s3), vec(b3))


def _sc_sparse_body(xyz_hbm, h_hbm, npts_hbm, g_hbm,
                    x_v, y_v, z_v, mind_v, fps_v, bidx_v, npts_v, rows_v, sem):
    cix = lax.axis_index("c")
    six = lax.axis_index("s")
    wid = six * 2 + cix
    b = wid // _WQ
    q = wid % _WQ

    iota = lax.iota(jnp.int32, _L)
    zeros16 = jnp.full((_L,), b * 0, jnp.int32)
    lane0 = iota == 0

    pltpu.sync_copy(xyz_hbm.at[pl.ds(b * 3 * _NN, _NN)], x_v)
    pltpu.sync_copy(xyz_hbm.at[pl.ds(b * 3 * _NN + _NN, _NN)], y_v)
    pltpu.sync_copy(xyz_hbm.at[pl.ds(b * 3 * _NN + 2 * _NN, _NN)], z_v)

    big = jnp.full((_L,), 1e10, jnp.float32)

    def init_body(j, carry):
        mind_v[pl.ds(j * _L, _L)] = big
        return carry
    lax.fori_loop(0, _NN // _L, init_body, 0)

    def initf_body(j, carry):
        fps_v[pl.ds(j * _L, _L)] = iota * 0
        return carry
    lax.fori_loop(0, M_CENTROIDS // _L, initf_body, 0)
    def initb_body(j, carry):
        bidx_v[pl.ds(j * _L, _L)] = iota * 0 + b * _NN
        return carry
    lax.fori_loop(0, _MW * KNBR // _L, initb_body, 0)
    xv0 = x_v[pl.ds(0, _L)]
    yv0 = y_v[pl.ds(0, _L)]
    zv0 = z_v[pl.ds(0, _L)]
    nbig = jnp.float32(-3.4e38)
    lastx = jnp.full((_L,), jnp.max(jnp.where(lane0, xv0, nbig)), jnp.float32)
    lasty = jnp.full((_L,), jnp.max(jnp.where(lane0, yv0, nbig)), jnp.float32)
    lastz = jnp.full((_L,), jnp.max(jnp.where(lane0, zv0, nbig)), jnp.float32)
    neginf = jnp.full((_L,), -3.4e38, jnp.float32)

    def fps_step(i, carry):
        lx, ly, lz = carry

        def sweep(jj, c2):
            bestd, besti = c2
            for u in range(4):
                off = jj * (4 * _L) + u * _L
                xv = x_v[pl.ds(off, _L)]
                yv = y_v[pl.ds(off, _L)]
                zv = z_v[pl.ds(off, _L)]
                dx = xv - lx
                dy = yv - ly
                dz = zv - lz
                d2 = (dx * dx + dy * dy) + dz * dz
                md = jnp.minimum(mind_v[pl.ds(off, _L)], d2)
                mind_v[pl.ds(off, _L)] = md
                upd = md > bestd
                bestd = jnp.where(upd, md, bestd)
                besti = jnp.where(upd, iota + off, besti)
            return bestd, besti

        bestd, besti = lax.fori_loop(0, _NN // (4 * _L), sweep,
                                     (neginf, zeros16))
        mx = jnp.max(bestd)
        cand = jnp.where(bestd == mx, besti, _NN)
        nxt = jnp.min(cand)
        nxtv = jnp.full((_L,), nxt, jnp.int32)
        plsc.store_scatter(fps_v, [jnp.full((_L,), i, jnp.int32)], nxtv,
                           mask=lane0)
        return (plsc.load_gather(x_v, [nxtv]),
                plsc.load_gather(y_v, [nxtv]),
                plsc.load_gather(z_v, [nxtv]))

    lax.fori_loop(1, M_CENTROIDS, fps_step, (lastx, lasty, lastz))

    for g in range(_MW // _L):
        cidxv = fps_v[pl.ds(q * _MW + g * _L, _L)]
        cx = plsc.load_gather(x_v, [cidxv])
        cy = plsc.load_gather(y_v, [cidxv])
        cz = plsc.load_gather(z_v, [cidxv])
        rowbase = (g * _L + iota) * 3
        plsc.store_scatter(npts_v, [rowbase], cx)
        plsc.store_scatter(npts_v, [rowbase + 1], cy)
        plsc.store_scatter(npts_v, [rowbase + 2], cz)
    pltpu.sync_copy(
        npts_v,
        npts_hbm.at[pl.ds(b * M_CENTROIDS * 3 + q * _MW * 3, _MW * 3)])

    r2 = jnp.float32(RADIUS * RADIUS)

    def ball_one(cm, carry):
        pos_all = q * _MW + cm
        vbase = (pos_all // _L) * _L
        lane = pos_all % _L
        grp = fps_v[pl.ds(vbase, _L)]
        cs = jnp.max(jnp.where(iota == lane, grp, -1))
        csv = jnp.full((_L,), cs, jnp.int32)
        cx = plsc.load_gather(x_v, [csv])
        cy = plsc.load_gather(y_v, [csv])
        cz = plsc.load_gather(z_v, [csv])
        base_b = cm * KNBR

        def scan(j, cnt):
            off = j * _L
            dx = x_v[pl.ds(off, _L)] - cx
            dy = y_v[pl.ds(off, _L)] - cy
            dz = z_v[pl.ds(off, _L)] - cz
            d2 = (dx * dx + dy * dy) + dz * dz
            msk = d2 < r2
            cums = plsc.cumsum(msk.astype(jnp.int32))
            pos = cnt + cums - 1
            wm = msk & (pos < KNBR)
            plsc.store_scatter(bidx_v, [base_b + pos],
                               iota + (off + b * _NN), mask=wm)
            pc = plsc.all_reduce_population_count(msk)
            return cnt + pc

        cnt = lax.fori_loop(0, _NN // _L, scan, zeros16)

        firstv = plsc.load_gather(bidx_v, [jnp.full((_L,), base_b, jnp.int32)])
        for u in range(KNBR // _L):
            sl = pl.ds(base_b + u * _L, _L)
            cur = bidx_v[sl]
            lanepos = iota + u * _L
            bidx_v[sl] = jnp.where(lanepos < cnt, cur, firstv)
        return carry

    lax.fori_loop(0, _MW, ball_one, 0)

    for i in range(0):
        idx_sl = bidx_v.at[pl.ds(i * _GCHUNK, _GCHUNK)]
        pltpu.async_copy(h_hbm.at[idx_sl], rows_v, sem).wait()
        row0 = (b * M_CENTROIDS + q * _MW) * KNBR + i * _GCHUNK
        pltpu.sync_copy(rows_v, g_hbm.at[pl.ds(row0, _GCHUNK), :])


def _sc_sparse(xyz, H):
    B, _, N = xyz.shape
    mesh = plsc.VectorSubcoreMesh(core_axis_name="c", subcore_axis_name="s")
    f = pl.kernel(
        _sc_sparse_body,
        mesh=mesh,
        compiler_params=pltpu.CompilerParams(needs_layout_passes=False),
        out_type=[
            jax.ShapeDtypeStruct((B * M_CENTROIDS * 3,), jnp.float32),
            jax.ShapeDtypeStruct((B * M_CENTROIDS * KNBR, 128), jnp.float32),
        ],
        scratch_types=[
            pltpu.VMEM((N,), jnp.float32),
            pltpu.VMEM((N,), jnp.float32),
            pltpu.VMEM((N,), jnp.float32),
            pltpu.VMEM((N,), jnp.float32),
            pltpu.VMEM((M_CENTROIDS,), jnp.int32),
            pltpu.VMEM((_MW * KNBR,), jnp.int32),
            pltpu.VMEM((_MW * 3,), jnp.float32),
            pltpu.VMEM((_GCHUNK, 128), jnp.float32),
            pltpu.SemaphoreType.DMA,
        ],
    )
    return f(xyz.reshape(-1), H.reshape(-1, 128))


def _gather_rows(x, idx):
    return jax.vmap(lambda xb, ib: xb[ib])(x, idx)


def _fps_scaffold(pts, M):
    B, N, _ = pts.shape
    def body(i, state):
        idxs, min_d, last = state
        last_pt = _gather_rows(pts, last)
        d = jnp.sum((pts - last_pt[:, None, :]) ** 2, axis=-1)
        min_d = jnp.minimum(min_d, d)
        nxt = jnp.argmax(min_d, axis=-1).astype(jnp.int32)
        idxs = idxs.at[:, i].set(nxt)
        return (idxs, min_d, nxt)
    state0 = (jnp.zeros((B, M), jnp.int32),
              jnp.full((B, N), 1e10, jnp.float32),
              jnp.zeros((B,), jnp.int32))
    idxs, _, _ = lax.fori_loop(1, M, body, state0)
    return idxs


def _ball_scaffold(new_pts, pts, radius, K):
    d2 = jnp.sum((new_pts[:, :, None, :] - pts[:, None, :, :]) ** 2, axis=-1)
    N = pts.shape[1]
    cand = jnp.where(d2 < radius * radius,
                     jnp.arange(N, dtype=jnp.int32)[None, None, :], N)
    idx_sorted = jnp.sort(cand, axis=-1)[:, :, :K]
    first = idx_sorted[:, :, 0:1]
    idx = jnp.where(idx_sorted < N, idx_sorted, first)
    return jnp.minimum(idx, N - 1).astype(jnp.int32)


def kernel(xyz, feature, W1, gamma1, beta1, W2, gamma2, beta2, W3, gamma3, beta3):
    B, _, N = xyz.shape
    inv = 1.0 / jnp.sqrt(1.0 + EPS_BN)
    s1, s2, s3 = gamma1 * inv, gamma2 * inv, gamma3 * inv

    ptsT = jnp.transpose(xyz, (0, 2, 1))
    featT = jnp.transpose(feature, (0, 2, 1))
    w1xT = jnp.transpose(W1[:, :3])
    w1fT = jnp.transpose(W1[:, 3:])
    w2T = jnp.transpose(W2)
    w3T = jnp.transpose(W3)

    H = _prep_h(ptsT, featT, w1xT, w1fT)

    npts_flat, G_flat = _sc_sparse(xyz, H)
    new_pts = npts_flat.reshape(B, M_CENTROIDS, 3)
    G = G_flat.reshape(B, M_CENTROIDS, KNBR, 128)

    nf = _mlp_maxpool(G, new_pts, w1xT, w2T, w3T, s1, beta1, s2, beta2, s3, beta3)
    new_xyz = jnp.transpose(new_pts, (0, 2, 1))
    new_feature = jnp.transpose(nf, (0, 2, 1))
    return (new_xyz, new_feature)


# speedup vs baseline: 2.0898x; 1.1357x over previous
import functools
import jax
import jax.numpy as jnp
from jax import lax
from jax.experimental import pallas as pl
from jax.experimental.pallas import tpu as pltpu
from jax.experimental.pallas import tpu_sc as plsc

M_CENTROIDS = 512
RADIUS = 0.15
KNBR = 32
EPS_BN = 1e-5
_NB = 8
_NN = 2048
_L = 16
_NW = 32
_WQ = _NW // _NB
_MW = M_CENTROIDS // _WQ
_GCHUNK = 256


def _prep_body(ptsT_ref, featT_ref, w1xT_ref, w1fT_ref, h_ref):
    ptsT = ptsT_ref[0]
    featT = featT_ref[0]
    h = jnp.dot(featT, w1fT_ref[...], preferred_element_type=jnp.float32)
    h = h + jnp.dot(ptsT, w1xT_ref[...], preferred_element_type=jnp.float32)
    h_ref[0] = h


def _prep_h(ptsT, featT, w1xT, w1fT):
    B, N, _ = ptsT.shape
    return pl.pallas_call(
        _prep_body,
        grid=(B,),
        in_specs=[
            pl.BlockSpec((1, N, 3), lambda b: (b, 0, 0)),
            pl.BlockSpec((1, N, featT.shape[2]), lambda b: (b, 0, 0)),
            pl.BlockSpec(w1xT.shape, lambda b: (0, 0)),
            pl.BlockSpec(w1fT.shape, lambda b: (0, 0)),
        ],
        out_specs=pl.BlockSpec((1, N, 128), lambda b: (b, 0, 0)),
        out_shape=jax.ShapeDtypeStruct((B, N, 128), jnp.float32),
    )(ptsT, featT, w1xT, w1fT)


def _mlp_body(g_ref, nxT_ref, w1xT_ref, w2T_ref, w3T_ref,
              s1_ref, b1_ref, s2_ref, b2_ref, s3_ref, b3_ref, out_ref):
    Mt = g_ref.shape[1]
    g = g_ref[0]
    nx = nxT_ref[0]
    cc = jnp.dot(nx, w1xT_ref[...], preferred_element_type=jnp.float32)
    y1 = g - cc[:, None, :]
    y1 = jnp.maximum(y1 * s1_ref[0][None, None, :] + b1_ref[0][None, None, :], 0.0)
    x1 = y1.reshape(Mt * KNBR, 128)
    y2 = jnp.dot(x1, w2T_ref[...], preferred_element_type=jnp.float32)
    y2 = jnp.maximum(y2 * s2_ref[0][None, :] + b2_ref[0][None, :], 0.0)
    y3 = jnp.dot(y2, w3T_ref[...], preferred_element_type=jnp.float32)
    y3 = jnp.maximum(y3 * s3_ref[0][None, :] + b3_ref[0][None, :], 0.0)
    m3 = y3.reshape(Mt, KNBR, 256)
    r = m3[:, 0, :]
    for k in range(1, KNBR):
        r = jnp.maximum(r, m3[:, k, :])
    out_ref[0] = r


def _mlp_maxpool(G, new_ptsT, w1xT, w2T, w3T, s1, b1, s2, b2, s3, b3):
    B, M, K, _ = G.shape
    Mt = 128
    vec = lambda v: v.reshape(1, -1)
    return pl.pallas_call(
        _mlp_body,
        grid=(B, M // Mt),
        in_specs=[
            pl.BlockSpec((1, Mt, K, 128), lambda b, m: (b, m, 0, 0)),
            pl.BlockSpec((1, Mt, 3), lambda b, m: (b, m, 0)),
            pl.BlockSpec((3, 128), lambda b, m: (0, 0)),
            pl.BlockSpec((128, 128), lambda b, m: (0, 0)),
            pl.BlockSpec((128, 256), lambda b, m: (0, 0)),
            pl.BlockSpec((1, 128), lambda b, m: (0, 0)),
            pl.BlockSpec((1, 128), lambda b, m: (0, 0)),
            pl.BlockSpec((1, 128), lambda b, m: (0, 0)),
            pl.BlockSpec((1, 128), lambda b, m: (0, 0)),
            pl.BlockSpec((1, 256), lambda b, m: (0, 0)),
            pl.BlockSpec((1, 256), lambda b, m: (0, 0)),
        ],
        out_specs=pl.BlockSpec((1, Mt, 256), lambda b, m: (b, m, 0)),
        out_shape=jax.ShapeDtypeStruct((B, M, 256), jnp.float32),
    )(G, new_ptsT, w1xT, w2T, w3T,
      vec(s1), vec(b1), vec(s2), vec(b2), vec(s3), vec(b3))


def _sc_sparse_body(xyz_hbm, h_hbm, npts_hbm, g_hbm,
                    x_v, y_v, z_v, mind_v, fps_v, bidx_v, npts_v, rows_v, sem):
    cix = lax.axis_index("c")
    six = lax.axis_index("s")
    wid = six * 2 + cix
    b = wid // _WQ
    q = wid % _WQ

    iota = lax.iota(jnp.int32, _L)
    zeros16 = jnp.full((_L,), b * 0, jnp.int32)
    lane0 = iota == 0

    pltpu.sync_copy(xyz_hbm.at[pl.ds(b * 3 * _NN, _NN)], x_v)
    pltpu.sync_copy(xyz_hbm.at[pl.ds(b * 3 * _NN + _NN, _NN)], y_v)
    pltpu.sync_copy(xyz_hbm.at[pl.ds(b * 3 * _NN + 2 * _NN, _NN)], z_v)

    big = jnp.full((_L,), 1e10, jnp.float32)

    def init_body(j, carry):
        mind_v[pl.ds(j * _L, _L)] = big
        return carry
    lax.fori_loop(0, _NN // _L, init_body, 0)

    def initf_body(j, carry):
        fps_v[pl.ds(j * _L, _L)] = iota * 0
        return carry
    lax.fori_loop(0, M_CENTROIDS // _L, initf_body, 0)
    def initb_body(j, carry):
        bidx_v[pl.ds(j * _L, _L)] = iota * 0 + b * _NN
        return carry
    lax.fori_loop(0, _MW * KNBR // _L, initb_body, 0)
    xv0 = x_v[pl.ds(0, _L)]
    yv0 = y_v[pl.ds(0, _L)]
    zv0 = z_v[pl.ds(0, _L)]
    nbig = jnp.float32(-3.4e38)
    lastx = jnp.full((_L,), jnp.max(jnp.where(lane0, xv0, nbig)), jnp.float32)
    lasty = jnp.full((_L,), jnp.max(jnp.where(lane0, yv0, nbig)), jnp.float32)
    lastz = jnp.full((_L,), jnp.max(jnp.where(lane0, zv0, nbig)), jnp.float32)
    neginf = jnp.full((_L,), -3.4e38, jnp.float32)

    def fps_step(i, carry):
        lx, ly, lz = carry

        def sweep(jj, c2):
            bestd, besti = c2
            for u in range(4):
                off = jj * (4 * _L) + u * _L
                xv = x_v[pl.ds(off, _L)]
                yv = y_v[pl.ds(off, _L)]
                zv = z_v[pl.ds(off, _L)]
                dx = xv - lx
                dy = yv - ly
                dz = zv - lz
                d2 = (dx * dx + dy * dy) + dz * dz
                md = jnp.minimum(mind_v[pl.ds(off, _L)], d2)
                mind_v[pl.ds(off, _L)] = md
                upd = md > bestd
                bestd = jnp.where(upd, md, bestd)
                besti = jnp.where(upd, iota + off, besti)
            return bestd, besti

        bestd, besti = lax.fori_loop(0, _NN // (4 * _L), sweep,
                                     (neginf, zeros16))
        mx = jnp.max(bestd)
        cand = jnp.where(bestd == mx, besti, _NN)
        nxt = jnp.min(cand)
        nxtv = jnp.full((_L,), nxt, jnp.int32)
        plsc.store_scatter(fps_v, [jnp.full((_L,), i, jnp.int32)], nxtv,
                           mask=lane0)
        return (plsc.load_gather(x_v, [nxtv]),
                plsc.load_gather(y_v, [nxtv]),
                plsc.load_gather(z_v, [nxtv]))

    lax.fori_loop(1, M_CENTROIDS, fps_step, (lastx, lasty, lastz))

    for g in range(_MW // _L):
        cidxv = fps_v[pl.ds(q * _MW + g * _L, _L)]
        cx = plsc.load_gather(x_v, [cidxv])
        cy = plsc.load_gather(y_v, [cidxv])
        cz = plsc.load_gather(z_v, [cidxv])
        rowbase = (g * _L + iota) * 3
        plsc.store_scatter(npts_v, [rowbase], cx)
        plsc.store_scatter(npts_v, [rowbase + 1], cy)
        plsc.store_scatter(npts_v, [rowbase + 2], cz)
    pltpu.sync_copy(
        npts_v,
        npts_hbm.at[pl.ds(b * M_CENTROIDS * 3 + q * _MW * 3, _MW * 3)])

    r2 = jnp.float32(RADIUS * RADIUS)

    def ball_one(cm, carry):
        pos_all = q * _MW + cm
        vbase = (pos_all // _L) * _L
        lane = pos_all % _L
        grp = fps_v[pl.ds(vbase, _L)]
        cs = jnp.max(jnp.where(iota == lane, grp, -1))
        csv = jnp.full((_L,), cs, jnp.int32)
        cx = plsc.load_gather(x_v, [csv])
        cy = plsc.load_gather(y_v, [csv])
        cz = plsc.load_gather(z_v, [csv])
        base_b = cm * KNBR

        def scan(j, cnt):
            msks = []
            cumss = []
            pcs = []
            for u in range(4):
                off = j * (4 * _L) + u * _L
                dx = x_v[pl.ds(off, _L)] - cx
                dy = y_v[pl.ds(off, _L)] - cy
                dz = z_v[pl.ds(off, _L)] - cz
                d2 = (dx * dx + dy * dy) + dz * dz
                msk = d2 < r2
                msks.append(msk)
                cumss.append(plsc.cumsum(msk.astype(jnp.int32)))
                pcs.append(plsc.all_reduce_population_count(msk))
            for u in range(4):
                off = j * (4 * _L) + u * _L
                pos = cnt + cumss[u] - 1
                wm = msks[u] & (pos < KNBR)
                plsc.store_scatter(bidx_v, [base_b + pos],
                                   iota + (off + b * _NN), mask=wm)
                cnt = cnt + pcs[u]
            return cnt

        cnt = lax.fori_loop(0, _NN // (4 * _L), scan, zeros16)

        firstv = plsc.load_gather(bidx_v, [jnp.full((_L,), base_b, jnp.int32)])
        for u in range(KNBR // _L):
            sl = pl.ds(base_b + u * _L, _L)
            cur = bidx_v[sl]
            lanepos = iota + u * _L
            bidx_v[sl] = jnp.where(lanepos < cnt, cur, firstv)
        return carry

    lax.fori_loop(0, _MW, ball_one, 0)

    for i in range(_MW * KNBR // _GCHUNK):
        idx_sl = bidx_v.at[pl.ds(i * _GCHUNK, _GCHUNK)]
        pltpu.async_copy(h_hbm.at[idx_sl], rows_v, sem).wait()
        row0 = (b * M_CENTROIDS + q * _MW) * KNBR + i * _GCHUNK
        pltpu.sync_copy(rows_v, g_hbm.at[pl.ds(row0, _GCHUNK), :])


def _sc_sparse(xyz, H):
    B, _, N = xyz.shape
    mesh = plsc.VectorSubcoreMesh(core_axis_name="c", subcore_axis_name="s")
    f = pl.kernel(
        _sc_sparse_body,
        mesh=mesh,
        compiler_params=pltpu.CompilerParams(needs_layout_passes=False),
        out_type=[
            jax.ShapeDtypeStruct((B * M_CENTROIDS * 3,), jnp.float32),
            jax.ShapeDtypeStruct((B * M_CENTROIDS * KNBR, 128), jnp.float32),
        ],
        scratch_types=[
            pltpu.VMEM((N,), jnp.float32),
            pltpu.VMEM((N,), jnp.float32),
            pltpu.VMEM((N,), jnp.float32),
            pltpu.VMEM((N,), jnp.float32),
            pltpu.VMEM((M_CENTROIDS,), jnp.int32),
            pltpu.VMEM((_MW * KNBR,), jnp.int32),
            pltpu.VMEM((_MW * 3,), jnp.float32),
            pltpu.VMEM((_GCHUNK, 128), jnp.float32),
            pltpu.SemaphoreType.DMA,
        ],
    )
    return f(xyz.reshape(-1), H.reshape(-1, 128))


def _gather_rows(x, idx):
    return jax.vmap(lambda xb, ib: xb[ib])(x, idx)


def _fps_scaffold(pts, M):
    B, N, _ = pts.shape
    def body(i, state):
        idxs, min_d, last = state
        last_pt = _gather_rows(pts, last)
        d = jnp.sum((pts - last_pt[:, None, :]) ** 2, axis=-1)
        min_d = jnp.minimum(min_d, d)
        nxt = jnp.argmax(min_d, axis=-1).astype(jnp.int32)
        idxs = idxs.at[:, i].set(nxt)
        return (idxs, min_d, nxt)
    state0 = (jnp.zeros((B, M), jnp.int32),
              jnp.full((B, N), 1e10, jnp.float32),
              jnp.zeros((B,), jnp.int32))
    idxs, _, _ = lax.fori_loop(1, M, body, state0)
    return idxs


def _ball_scaffold(new_pts, pts, radius, K):
    d2 = jnp.sum((new_pts[:, :, None, :] - pts[:, None, :, :]) ** 2, axis=-1)
    N = pts.shape[1]
    cand = jnp.where(d2 < radius * radius,
                     jnp.arange(N, dtype=jnp.int32)[None, None, :], N)
    idx_sorted = jnp.sort(cand, axis=-1)[:, :, :K]
    first = idx_sorted[:, :, 0:1]
    idx = jnp.where(idx_sorted < N, idx_sorted, first)
    return jnp.minimum(idx, N - 1).astype(jnp.int32)


def kernel(xyz, feature, W1, gamma1, beta1, W2, gamma2, beta2, W3, gamma3, beta3):
    B, _, N = xyz.shape
    inv = 1.0 / jnp.sqrt(1.0 + EPS_BN)
    s1, s2, s3 = gamma1 * inv, gamma2 * inv, gamma3 * inv

    ptsT = jnp.transpose(xyz, (0, 2, 1))
    featT = jnp.transpose(feature, (0, 2, 1))
    w1xT = jnp.transpose(W1[:, :3])
    w1fT = jnp.transpose(W1[:, 3:])
    w2T = jnp.transpose(W2)
    w3T = jnp.transpose(W3)

    H = _prep_h(ptsT, featT, w1xT, w1fT)

    npts_flat, G_flat = _sc_sparse(xyz, H)
    new_pts = npts_flat.reshape(B, M_CENTROIDS, 3)
    G = G_flat.reshape(B, M_CENTROIDS, KNBR, 128)

    nf = _mlp_maxpool(G, new_pts, w1xT, w2T, w3T, s1, beta1, s2, beta2, s3, beta3)
    new_xyz = jnp.transpose(new_pts, (0, 2, 1))
    new_feature = jnp.transpose(nf, (0, 2, 1))
    return (new_xyz, new_feature)


# speedup vs baseline: 2.1546x; 1.0310x over previous
import functools
import jax
import jax.numpy as jnp
from jax import lax
from jax.experimental import pallas as pl
from jax.experimental.pallas import tpu as pltpu
from jax.experimental.pallas import tpu_sc as plsc

M_CENTROIDS = 512
RADIUS = 0.15
KNBR = 32
EPS_BN = 1e-5
_NB = 8
_NN = 2048
_L = 16
_NW = 32
_WQ = _NW // _NB
_MW = M_CENTROIDS // _WQ
_GCHUNK = 256


def _prep_body(ptsT_ref, featT_ref, w1xT_ref, w1fT_ref, h_ref):
    ptsT = ptsT_ref[0]
    featT = featT_ref[0]
    h = jnp.dot(featT, w1fT_ref[...], preferred_element_type=jnp.float32)
    h = h + jnp.dot(ptsT, w1xT_ref[...], preferred_element_type=jnp.float32)
    h_ref[0] = h


def _prep_h(ptsT, featT, w1xT, w1fT):
    B, N, _ = ptsT.shape
    return pl.pallas_call(
        _prep_body,
        grid=(B,),
        in_specs=[
            pl.BlockSpec((1, N, 3), lambda b: (b, 0, 0)),
            pl.BlockSpec((1, N, featT.shape[2]), lambda b: (b, 0, 0)),
            pl.BlockSpec(w1xT.shape, lambda b: (0, 0)),
            pl.BlockSpec(w1fT.shape, lambda b: (0, 0)),
        ],
        out_specs=pl.BlockSpec((1, N, 128), lambda b: (b, 0, 0)),
        out_shape=jax.ShapeDtypeStruct((B, N, 128), jnp.float32),
    )(ptsT, featT, w1xT, w1fT)


def _mlp_body(g_ref, nxT_ref, w1xT_ref, w2T_ref, w3T_ref,
              s1_ref, b1_ref, s2_ref, b2_ref, s3_ref, b3_ref, out_ref):
    Mt = g_ref.shape[1]
    g = g_ref[0]
    nx = nxT_ref[0]
    cc = jnp.dot(nx, w1xT_ref[...], preferred_element_type=jnp.float32)
    y1 = g - cc[:, None, :]
    y1 = jnp.maximum(y1 * s1_ref[0][None, None, :] + b1_ref[0][None, None, :], 0.0)
    x1 = y1.reshape(Mt * KNBR, 128)
    y2 = jnp.dot(x1, w2T_ref[...], preferred_element_type=jnp.float32)
    y2 = jnp.maximum(y2 * s2_ref[0][None, :] + b2_ref[0][None, :], 0.0)
    y3 = jnp.dot(y2, w3T_ref[...], preferred_element_type=jnp.float32)
    y3 = jnp.maximum(y3 * s3_ref[0][None, :] + b3_ref[0][None, :], 0.0)
    m3 = y3.reshape(Mt, KNBR, 256)
    r = m3[:, 0, :]
    for k in range(1, KNBR):
        r = jnp.maximum(r, m3[:, k, :])
    out_ref[0] = r


def _mlp_maxpool(G, new_ptsT, w1xT, w2T, w3T, s1, b1, s2, b2, s3, b3):
    B, M, K, _ = G.shape
    Mt = 128
    vec = lambda v: v.reshape(1, -1)
    return pl.pallas_call(
        _mlp_body,
        grid=(B, M // Mt),
        in_specs=[
            pl.BlockSpec((1, Mt, K, 128), lambda b, m: (b, m, 0, 0)),
            pl.BlockSpec((1, Mt, 3), lambda b, m: (b, m, 0)),
            pl.BlockSpec((3, 128), lambda b, m: (0, 0)),
            pl.BlockSpec((128, 128), lambda b, m: (0, 0)),
            pl.BlockSpec((128, 256), lambda b, m: (0, 0)),
            pl.BlockSpec((1, 128), lambda b, m: (0, 0)),
            pl.BlockSpec((1, 128), lambda b, m: (0, 0)),
            pl.BlockSpec((1, 128), lambda b, m: (0, 0)),
            pl.BlockSpec((1, 128), lambda b, m: (0, 0)),
            pl.BlockSpec((1, 256), lambda b, m: (0, 0)),
            pl.BlockSpec((1, 256), lambda b, m: (0, 0)),
        ],
        out_specs=pl.BlockSpec((1, Mt, 256), lambda b, m: (b, m, 0)),
        out_shape=jax.ShapeDtypeStruct((B, M, 256), jnp.float32),
    )(G, new_ptsT, w1xT, w2T, w3T,
      vec(s1), vec(b1), vec(s2), vec(b2), vec(s3), vec(b3))


def _sc_sparse_body(xyz_hbm, h_hbm, npts_hbm, g_hbm,
                    x_v, y_v, z_v, mind_v, fps_v, bidx_v, npts_v, rows_v, sem):
    cix = lax.axis_index("c")
    six = lax.axis_index("s")
    wid = six * 2 + cix
    b = wid // _WQ
    q = wid % _WQ

    iota = lax.iota(jnp.int32, _L)
    zeros16 = jnp.full((_L,), b * 0, jnp.int32)
    lane0 = iota == 0

    pltpu.sync_copy(xyz_hbm.at[pl.ds(b * 3 * _NN, _NN)], x_v)
    pltpu.sync_copy(xyz_hbm.at[pl.ds(b * 3 * _NN + _NN, _NN)], y_v)
    pltpu.sync_copy(xyz_hbm.at[pl.ds(b * 3 * _NN + 2 * _NN, _NN)], z_v)

    big = jnp.full((_L,), 1e10, jnp.float32)

    def init_body(j, carry):
        mind_v[pl.ds(j * _L, _L)] = big
        return carry
    lax.fori_loop(0, _NN // _L, init_body, 0)

    def initf_body(j, carry):
        fps_v[pl.ds(j * _L, _L)] = iota * 0
        return carry
    lax.fori_loop(0, M_CENTROIDS // _L, initf_body, 0)
    def initb_body(j, carry):
        bidx_v[pl.ds(j * _L, _L)] = iota * 0 + b * _NN
        return carry
    lax.fori_loop(0, _MW * KNBR // _L, initb_body, 0)
    xv0 = x_v[pl.ds(0, _L)]
    yv0 = y_v[pl.ds(0, _L)]
    zv0 = z_v[pl.ds(0, _L)]
    nbig = jnp.float32(-3.4e38)
    lastx = jnp.full((_L,), jnp.max(jnp.where(lane0, xv0, nbig)), jnp.float32)
    lasty = jnp.full((_L,), jnp.max(jnp.where(lane0, yv0, nbig)), jnp.float32)
    lastz = jnp.full((_L,), jnp.max(jnp.where(lane0, zv0, nbig)), jnp.float32)
    neginf = jnp.full((_L,), -3.4e38, jnp.float32)

    def fps_step(i, carry):
        lx, ly, lz = carry

        def sweep(jj, c2):
            bestd, besti = c2
            for u in range(4):
                off = jj * (4 * _L) + u * _L
                xv = x_v[pl.ds(off, _L)]
                yv = y_v[pl.ds(off, _L)]
                zv = z_v[pl.ds(off, _L)]
                dx = xv - lx
                dy = yv - ly
                dz = zv - lz
                d2 = (dx * dx + dy * dy) + dz * dz
                md = jnp.minimum(mind_v[pl.ds(off, _L)], d2)
                mind_v[pl.ds(off, _L)] = md
                upd = md > bestd
                bestd = jnp.where(upd, md, bestd)
                besti = jnp.where(upd, iota + off, besti)
            return bestd, besti

        bestd, besti = lax.fori_loop(0, _NN // (4 * _L), sweep,
                                     (neginf, zeros16))
        mx = jnp.max(bestd)
        cand = jnp.where(bestd == mx, besti, _NN)
        nxt = jnp.min(cand)
        nxtv = jnp.full((_L,), nxt, jnp.int32)
        plsc.store_scatter(fps_v, [jnp.full((_L,), i, jnp.int32)], nxtv,
                           mask=lane0)
        return (plsc.load_gather(x_v, [nxtv]),
                plsc.load_gather(y_v, [nxtv]),
                plsc.load_gather(z_v, [nxtv]))

    lax.fori_loop(1, M_CENTROIDS, fps_step, (lastx, lasty, lastz))

    for g in range(_MW // _L):
        cidxv = fps_v[pl.ds(q * _MW + g * _L, _L)]
        cx = plsc.load_gather(x_v, [cidxv])
        cy = plsc.load_gather(y_v, [cidxv])
        cz = plsc.load_gather(z_v, [cidxv])
        rowbase = (g * _L + iota) * 3
        plsc.store_scatter(npts_v, [rowbase], cx)
        plsc.store_scatter(npts_v, [rowbase + 1], cy)
        plsc.store_scatter(npts_v, [rowbase + 2], cz)
    pltpu.sync_copy(
        npts_v,
        npts_hbm.at[pl.ds(b * M_CENTROIDS * 3 + q * _MW * 3, _MW * 3)])

    r2 = jnp.float32(RADIUS * RADIUS)

    def ball_one(cm, carry):
        pos_all = q * _MW + cm
        vbase = (pos_all // _L) * _L
        lane = pos_all % _L
        grp = fps_v[pl.ds(vbase, _L)]
        cs = jnp.max(jnp.where(iota == lane, grp, -1))
        csv = jnp.full((_L,), cs, jnp.int32)
        cx = plsc.load_gather(x_v, [csv])
        cy = plsc.load_gather(y_v, [csv])
        cz = plsc.load_gather(z_v, [csv])
        base_b = cm * KNBR

        def scan(j, cnt):
            msks = []
            cumss = []
            pcs = []
            for u in range(8):
                off = j * (8 * _L) + u * _L
                dx = x_v[pl.ds(off, _L)] - cx
                dy = y_v[pl.ds(off, _L)] - cy
                dz = z_v[pl.ds(off, _L)] - cz
                d2 = (dx * dx + dy * dy) + dz * dz
                msk = d2 < r2
                msks.append(msk)
                cumss.append(plsc.cumsum(msk.astype(jnp.int32)))
                pcs.append(plsc.all_reduce_population_count(msk))
            for u in range(8):
                off = j * (8 * _L) + u * _L
                pos = cnt + cumss[u] - 1
                wm = msks[u] & (pos < KNBR)
                plsc.store_scatter(bidx_v, [base_b + pos],
                                   iota + (off + b * _NN), mask=wm)
                cnt = cnt + pcs[u]
            return cnt

        cnt = lax.fori_loop(0, _NN // (8 * _L), scan, zeros16)

        firstv = plsc.load_gather(bidx_v, [jnp.full((_L,), base_b, jnp.int32)])
        for u in range(KNBR // _L):
            sl = pl.ds(base_b + u * _L, _L)
            cur = bidx_v[sl]
            lanepos = iota + u * _L
            bidx_v[sl] = jnp.where(lanepos < cnt, cur, firstv)
        return carry

    lax.fori_loop(0, _MW, ball_one, 0)

    for i in range(_MW * KNBR // _GCHUNK):
        idx_sl = bidx_v.at[pl.ds(i * _GCHUNK, _GCHUNK)]
        pltpu.async_copy(h_hbm.at[idx_sl], rows_v, sem).wait()
        row0 = (b * M_CENTROIDS + q * _MW) * KNBR + i * _GCHUNK
        pltpu.sync_copy(rows_v, g_hbm.at[pl.ds(row0, _GCHUNK), :])


def _sc_sparse(xyz, H):
    B, _, N = xyz.shape
    mesh = plsc.VectorSubcoreMesh(core_axis_name="c", subcore_axis_name="s")
    f = pl.kernel(
        _sc_sparse_body,
        mesh=mesh,
        compiler_params=pltpu.CompilerParams(needs_layout_passes=False),
        out_type=[
            jax.ShapeDtypeStruct((B * M_CENTROIDS * 3,), jnp.float32),
            jax.ShapeDtypeStruct((B * M_CENTROIDS * KNBR, 128), jnp.float32),
        ],
        scratch_types=[
            pltpu.VMEM((N,), jnp.float32),
            pltpu.VMEM((N,), jnp.float32),
            pltpu.VMEM((N,), jnp.float32),
            pltpu.VMEM((N,), jnp.float32),
            pltpu.VMEM((M_CENTROIDS,), jnp.int32),
            pltpu.VMEM((_MW * KNBR,), jnp.int32),
            pltpu.VMEM((_MW * 3,), jnp.float32),
            pltpu.VMEM((_GCHUNK, 128), jnp.float32),
            pltpu.SemaphoreType.DMA,
        ],
    )
    return f(xyz.reshape(-1), H.reshape(-1, 128))


def _gather_rows(x, idx):
    return jax.vmap(lambda xb, ib: xb[ib])(x, idx)


def _fps_scaffold(pts, M):
    B, N, _ = pts.shape
    def body(i, state):
        idxs, min_d, last = state
        last_pt = _gather_rows(pts, last)
        d = jnp.sum((pts - last_pt[:, None, :]) ** 2, axis=-1)
        min_d = jnp.minimum(min_d, d)
        nxt = jnp.argmax(min_d, axis=-1).astype(jnp.int32)
        idxs = idxs.at[:, i].set(nxt)
        return (idxs, min_d, nxt)
    state0 = (jnp.zeros((B, M), jnp.int32),
              jnp.full((B, N), 1e10, jnp.float32),
              jnp.zeros((B,), jnp.int32))
    idxs, _, _ = lax.fori_loop(1, M, body, state0)
    return idxs


def _ball_scaffold(new_pts, pts, radius, K):
    d2 = jnp.sum((new_pts[:, :, None, :] - pts[:, None, :, :]) ** 2, axis=-1)
    N = pts.shape[1]
    cand = jnp.where(d2 < radius * radius,
                     jnp.arange(N, dtype=jnp.int32)[None, None, :], N)
    idx_sorted = jnp.sort(cand, axis=-1)[:, :, :K]
    first = idx_sorted[:, :, 0:1]
    idx = jnp.where(idx_sorted < N, idx_sorted, first)
    return jnp.minimum(idx, N - 1).astype(jnp.int32)


def kernel(xyz, feature, W1, gamma1, beta1, W2, gamma2, beta2, W3, gamma3, beta3):
    B, _, N = xyz.shape
    inv = 1.0 / jnp.sqrt(1.0 + EPS_BN)
    s1, s2, s3 = gamma1 * inv, gamma2 * inv, gamma3 * inv

    ptsT = jnp.transpose(xyz, (0, 2, 1))
    featT = jnp.transpose(feature, (0, 2, 1))
    w1xT = jnp.transpose(W1[:, :3])
    w1fT = jnp.transpose(W1[:, 3:])
    w2T = jnp.transpose(W2)
    w3T = jnp.transpose(W3)

    H = _prep_h(ptsT, featT, w1xT, w1fT)

    npts_flat, G_flat = _sc_sparse(xyz, H)
    new_pts = npts_flat.reshape(B, M_CENTROIDS, 3)
    G = G_flat.reshape(B, M_CENTROIDS, KNBR, 128)

    nf = _mlp_maxpool(G, new_pts, w1xT, w2T, w3T, s1, beta1, s2, beta2, s3, beta3)
    new_xyz = jnp.transpose(new_pts, (0, 2, 1))
    new_feature = jnp.transpose(nf, (0, 2, 1))
    return (new_xyz, new_feature)


# speedup vs baseline: 2.1681x; 1.0063x over previous
import functools
import jax
import jax.numpy as jnp
from jax import lax
from jax.experimental import pallas as pl
from jax.experimental.pallas import tpu as pltpu
from jax.experimental.pallas import tpu_sc as plsc

M_CENTROIDS = 512
RADIUS = 0.15
KNBR = 32
EPS_BN = 1e-5
_NB = 8
_NN = 2048
_L = 16
_NW = 32
_WQ = _NW // _NB
_MW = M_CENTROIDS // _WQ
_GCHUNK = 256


def _prep_body(ptsT_ref, featT_ref, w1xT_ref, w1fT_ref, h_ref):
    ptsT = ptsT_ref[0]
    featT = featT_ref[0]
    h = jnp.dot(featT, w1fT_ref[...], preferred_element_type=jnp.float32)
    h = h + jnp.dot(ptsT, w1xT_ref[...], preferred_element_type=jnp.float32)
    h_ref[0] = h


def _prep_h(ptsT, featT, w1xT, w1fT):
    B, N, _ = ptsT.shape
    return pl.pallas_call(
        _prep_body,
        grid=(B,),
        in_specs=[
            pl.BlockSpec((1, N, 3), lambda b: (b, 0, 0)),
            pl.BlockSpec((1, N, featT.shape[2]), lambda b: (b, 0, 0)),
            pl.BlockSpec(w1xT.shape, lambda b: (0, 0)),
            pl.BlockSpec(w1fT.shape, lambda b: (0, 0)),
        ],
        out_specs=pl.BlockSpec((1, N, 128), lambda b: (b, 0, 0)),
        out_shape=jax.ShapeDtypeStruct((B, N, 128), jnp.float32),
    )(ptsT, featT, w1xT, w1fT)


def _mlp_body(g_ref, nxT_ref, w1xT_ref, w2T_ref, w3T_ref,
              s1_ref, b1_ref, s2_ref, b2_ref, s3_ref, b3_ref, out_ref):
    Mt = g_ref.shape[1]
    g = g_ref[0]
    nx = nxT_ref[0]
    cc = jnp.dot(nx, w1xT_ref[...], preferred_element_type=jnp.float32)
    y1 = g - cc[:, None, :]
    y1 = jnp.maximum(y1 * s1_ref[0][None, None, :] + b1_ref[0][None, None, :], 0.0)
    x1 = y1.reshape(Mt * KNBR, 128)
    y2 = jnp.dot(x1, w2T_ref[...], preferred_element_type=jnp.float32)
    y2 = jnp.maximum(y2 * s2_ref[0][None, :] + b2_ref[0][None, :], 0.0)
    y3 = jnp.dot(y2, w3T_ref[...], preferred_element_type=jnp.float32)
    y3 = jnp.maximum(y3 * s3_ref[0][None, :] + b3_ref[0][None, :], 0.0)
    m3 = y3.reshape(Mt, KNBR, 256)
    r = m3[:, 0, :]
    for k in range(1, KNBR):
        r = jnp.maximum(r, m3[:, k, :])
    out_ref[0] = r


def _mlp_maxpool(G, new_ptsT, w1xT, w2T, w3T, s1, b1, s2, b2, s3, b3):
    B, M, K, _ = G.shape
    Mt = 128
    vec = lambda v: v.reshape(1, -1)
    return pl.pallas_call(
        _mlp_body,
        grid=(B, M // Mt),
        in_specs=[
            pl.BlockSpec((1, Mt, K, 128), lambda b, m: (b, m, 0, 0)),
            pl.BlockSpec((1, Mt, 3), lambda b, m: (b, m, 0)),
            pl.BlockSpec((3, 128), lambda b, m: (0, 0)),
            pl.BlockSpec((128, 128), lambda b, m: (0, 0)),
            pl.BlockSpec((128, 256), lambda b, m: (0, 0)),
            pl.BlockSpec((1, 128), lambda b, m: (0, 0)),
            pl.BlockSpec((1, 128), lambda b, m: (0, 0)),
            pl.BlockSpec((1, 128), lambda b, m: (0, 0)),
            pl.BlockSpec((1, 128), lambda b, m: (0, 0)),
            pl.BlockSpec((1, 256), lambda b, m: (0, 0)),
            pl.BlockSpec((1, 256), lambda b, m: (0, 0)),
        ],
        out_specs=pl.BlockSpec((1, Mt, 256), lambda b, m: (b, m, 0)),
        out_shape=jax.ShapeDtypeStruct((B, M, 256), jnp.float32),
    )(G, new_ptsT, w1xT, w2T, w3T,
      vec(s1), vec(b1), vec(s2), vec(b2), vec(s3), vec(b3))


def _sc_sparse_body(xyz_hbm, h_hbm, npts_hbm, g_hbm,
                    x_v, y_v, z_v, mind_v, fps_v, bidx_v, npts_v, rows_v, sem):
    cix = lax.axis_index("c")
    six = lax.axis_index("s")
    wid = six * 2 + cix
    b = wid // _WQ
    q = wid % _WQ

    iota = lax.iota(jnp.int32, _L)
    zeros16 = jnp.full((_L,), b * 0, jnp.int32)
    lane0 = iota == 0

    pltpu.sync_copy(xyz_hbm.at[pl.ds(b * 3 * _NN, _NN)], x_v)
    pltpu.sync_copy(xyz_hbm.at[pl.ds(b * 3 * _NN + _NN, _NN)], y_v)
    pltpu.sync_copy(xyz_hbm.at[pl.ds(b * 3 * _NN + 2 * _NN, _NN)], z_v)

    big = jnp.full((_L,), 1e10, jnp.float32)

    def init_body(j, carry):
        mind_v[pl.ds(j * _L, _L)] = big
        return carry
    lax.fori_loop(0, _NN // _L, init_body, 0)

    def initf_body(j, carry):
        fps_v[pl.ds(j * _L, _L)] = iota * 0
        return carry
    lax.fori_loop(0, M_CENTROIDS // _L, initf_body, 0)
    def initb_body(j, carry):
        bidx_v[pl.ds(j * _L, _L)] = iota * 0 + b * _NN
        return carry
    lax.fori_loop(0, _MW * KNBR // _L, initb_body, 0)
    xv0 = x_v[pl.ds(0, _L)]
    yv0 = y_v[pl.ds(0, _L)]
    zv0 = z_v[pl.ds(0, _L)]
    nbig = jnp.float32(-3.4e38)
    lastx = jnp.full((_L,), jnp.max(jnp.where(lane0, xv0, nbig)), jnp.float32)
    lasty = jnp.full((_L,), jnp.max(jnp.where(lane0, yv0, nbig)), jnp.float32)
    lastz = jnp.full((_L,), jnp.max(jnp.where(lane0, zv0, nbig)), jnp.float32)
    neginf = jnp.full((_L,), -3.4e38, jnp.float32)

    def fps_step(i, carry):
        lx, ly, lz = carry

        def sweep(jj, c2):
            bestd, besti = c2
            for u in range(4):
                off = jj * (4 * _L) + u * _L
                xv = x_v[pl.ds(off, _L)]
                yv = y_v[pl.ds(off, _L)]
                zv = z_v[pl.ds(off, _L)]
                dx = xv - lx
                dy = yv - ly
                dz = zv - lz
                d2 = (dx * dx + dy * dy) + dz * dz
                md = jnp.minimum(mind_v[pl.ds(off, _L)], d2)
                mind_v[pl.ds(off, _L)] = md
                upd = md > bestd
                bestd = jnp.where(upd, md, bestd)
                besti = jnp.where(upd, iota + off, besti)
            return bestd, besti

        bestd, besti = lax.fori_loop(0, _NN // (4 * _L), sweep,
                                     (neginf, zeros16))
        mx = jnp.max(bestd)
        cand = jnp.where(bestd == mx, besti, _NN)
        nxt = jnp.min(cand)
        nxtv = jnp.full((_L,), nxt, jnp.int32)
        plsc.store_scatter(fps_v, [jnp.full((_L,), i, jnp.int32)], nxtv,
                           mask=lane0)
        return (plsc.load_gather(x_v, [nxtv]),
                plsc.load_gather(y_v, [nxtv]),
                plsc.load_gather(z_v, [nxtv]))

    lax.fori_loop(1, M_CENTROIDS, fps_step, (lastx, lasty, lastz))

    for g in range(_MW // _L):
        cidxv = fps_v[pl.ds(q * _MW + g * _L, _L)]
        cx = plsc.load_gather(x_v, [cidxv])
        cy = plsc.load_gather(y_v, [cidxv])
        cz = plsc.load_gather(z_v, [cidxv])
        rowbase = (g * _L + iota) * 3
        plsc.store_scatter(npts_v, [rowbase], cx)
        plsc.store_scatter(npts_v, [rowbase + 1], cy)
        plsc.store_scatter(npts_v, [rowbase + 2], cz)
    pltpu.sync_copy(
        npts_v,
        npts_hbm.at[pl.ds(b * M_CENTROIDS * 3 + q * _MW * 3, _MW * 3)])

    r2 = jnp.float32(RADIUS * RADIUS)

    def ball_pair(cp, carry):
        cm_a = cp * 2
        pos_all = q * _MW + cm_a
        vbase = (pos_all // _L) * _L
        lane = pos_all % _L
        grp = fps_v[pl.ds(vbase, _L)]
        cs_a = jnp.max(jnp.where(iota == lane, grp, -1))
        cs_b = jnp.max(jnp.where(iota == lane + 1, grp, -1))
        csv_a = jnp.full((_L,), cs_a, jnp.int32)
        csv_b = jnp.full((_L,), cs_b, jnp.int32)
        cxa = plsc.load_gather(x_v, [csv_a])
        cya = plsc.load_gather(y_v, [csv_a])
        cza = plsc.load_gather(z_v, [csv_a])
        cxb = plsc.load_gather(x_v, [csv_b])
        cyb = plsc.load_gather(y_v, [csv_b])
        czb = plsc.load_gather(z_v, [csv_b])
        base_a = cm_a * KNBR
        base_bb = base_a + KNBR

        def scan(j, carry2):
            cnt_a, cnt_b = carry2
            ms = []
            for u in range(4):
                off = j * (4 * _L) + u * _L
                xv = x_v[pl.ds(off, _L)]
                yv = y_v[pl.ds(off, _L)]
                zv = z_v[pl.ds(off, _L)]
                dxa = xv - cxa
                dya = yv - cya
                dza = zv - cza
                d2a = (dxa * dxa + dya * dya) + dza * dza
                dxb = xv - cxb
                dyb = yv - cyb
                dzb = zv - czb
                d2b = (dxb * dxb + dyb * dyb) + dzb * dzb
                mska = d2a < r2
                mskb = d2b < r2
                ms.append((
                    mska, plsc.cumsum(mska.astype(jnp.int32)),
                    plsc.all_reduce_population_count(mska),
                    mskb, plsc.cumsum(mskb.astype(jnp.int32)),
                    plsc.all_reduce_population_count(mskb)))
            for u in range(4):
                off = j * (4 * _L) + u * _L
                mska, cuma, pca, mskb, cumb, pcb = ms[u]
                gidx = iota + (off + b * _NN)
                pos_a = cnt_a + cuma - 1
                wma = mska & (pos_a < KNBR)
                plsc.store_scatter(bidx_v, [base_a + pos_a], gidx, mask=wma)
                cnt_a = cnt_a + pca
                pos_b = cnt_b + cumb - 1
                wmb = mskb & (pos_b < KNBR)
                plsc.store_scatter(bidx_v, [base_bb + pos_b], gidx, mask=wmb)
                cnt_b = cnt_b + pcb
            return cnt_a, cnt_b

        cnt_a, cnt_b = lax.fori_loop(0, _NN // (4 * _L), scan,
                                     (zeros16, zeros16))

        for base_c, cnt in ((base_a, cnt_a), (base_bb, cnt_b)):
            firstv = plsc.load_gather(
                bidx_v, [jnp.full((_L,), base_c, jnp.int32)])
            for u in range(KNBR // _L):
                sl = pl.ds(base_c + u * _L, _L)
                cur = bidx_v[sl]
                lanepos = iota + u * _L
                bidx_v[sl] = jnp.where(lanepos < cnt, cur, firstv)
        return carry

    lax.fori_loop(0, _MW // 2, ball_pair, 0)

    for i in range(_MW * KNBR // _GCHUNK):
        idx_sl = bidx_v.at[pl.ds(i * _GCHUNK, _GCHUNK)]
        pltpu.async_copy(h_hbm.at[idx_sl], rows_v, sem).wait()
        row0 = (b * M_CENTROIDS + q * _MW) * KNBR + i * _GCHUNK
        pltpu.sync_copy(rows_v, g_hbm.at[pl.ds(row0, _GCHUNK), :])


def _sc_sparse(xyz, H):
    B, _, N = xyz.shape
    mesh = plsc.VectorSubcoreMesh(core_axis_name="c", subcore_axis_name="s")
    f = pl.kernel(
        _sc_sparse_body,
        mesh=mesh,
        compiler_params=pltpu.CompilerParams(needs_layout_passes=False),
        out_type=[
            jax.ShapeDtypeStruct((B * M_CENTROIDS * 3,), jnp.float32),
            jax.ShapeDtypeStruct((B * M_CENTROIDS * KNBR, 128), jnp.float32),
        ],
        scratch_types=[
            pltpu.VMEM((N,), jnp.float32),
            pltpu.VMEM((N,), jnp.float32),
            pltpu.VMEM((N,), jnp.float32),
            pltpu.VMEM((N,), jnp.float32),
            pltpu.VMEM((M_CENTROIDS,), jnp.int32),
            pltpu.VMEM((_MW * KNBR,), jnp.int32),
            pltpu.VMEM((_MW * 3,), jnp.float32),
            pltpu.VMEM((_GCHUNK, 128), jnp.float32),
            pltpu.SemaphoreType.DMA,
        ],
    )
    return f(xyz.reshape(-1), H.reshape(-1, 128))


def _gather_rows(x, idx):
    return jax.vmap(lambda xb, ib: xb[ib])(x, idx)


def _fps_scaffold(pts, M):
    B, N, _ = pts.shape
    def body(i, state):
        idxs, min_d, last = state
        last_pt = _gather_rows(pts, last)
        d = jnp.sum((pts - last_pt[:, None, :]) ** 2, axis=-1)
        min_d = jnp.minimum(min_d, d)
        nxt = jnp.argmax(min_d, axis=-1).astype(jnp.int32)
        idxs = idxs.at[:, i].set(nxt)
        return (idxs, min_d, nxt)
    state0 = (jnp.zeros((B, M), jnp.int32),
              jnp.full((B, N), 1e10, jnp.float32),
              jnp.zeros((B,), jnp.int32))
    idxs, _, _ = lax.fori_loop(1, M, body, state0)
    return idxs


def _ball_scaffold(new_pts, pts, radius, K):
    d2 = jnp.sum((new_pts[:, :, None, :] - pts[:, None, :, :]) ** 2, axis=-1)
    N = pts.shape[1]
    cand = jnp.where(d2 < radius * radius,
                     jnp.arange(N, dtype=jnp.int32)[None, None, :], N)
    idx_sorted = jnp.sort(cand, axis=-1)[:, :, :K]
    first = idx_sorted[:, :, 0:1]
    idx = jnp.where(idx_sorted < N, idx_sorted, first)
    return jnp.minimum(idx, N - 1).astype(jnp.int32)


def kernel(xyz, feature, W1, gamma1, beta1, W2, gamma2, beta2, W3, gamma3, beta3):
    B, _, N = xyz.shape
    inv = 1.0 / jnp.sqrt(1.0 + EPS_BN)
    s1, s2, s3 = gamma1 * inv, gamma2 * inv, gamma3 * inv

    ptsT = jnp.transpose(xyz, (0, 2, 1))
    featT = jnp.transpose(feature, (0, 2, 1))
    w1xT = jnp.transpose(W1[:, :3])
    w1fT = jnp.transpose(W1[:, 3:])
    w2T = jnp.transpose(W2)
    w3T = jnp.transpose(W3)

    H = _prep_h(ptsT, featT, w1xT, w1fT)

    npts_flat, G_flat = _sc_sparse(xyz, H)
    new_pts = npts_flat.reshape(B, M_CENTROIDS, 3)
    G = G_flat.reshape(B, M_CENTROIDS, KNBR, 128)

    nf = _mlp_maxpool(G, new_pts, w1xT, w2T, w3T, s1, beta1, s2, beta2, s3, beta3)
    new_xyz = jnp.transpose(new_pts, (0, 2, 1))
    new_feature = jnp.transpose(nf, (0, 2, 1))
    return (new_xyz, new_feature)


# speedup vs baseline: 2.2215x; 1.0246x over previous
import functools
import jax
import jax.numpy as jnp
from jax import lax
from jax.experimental import pallas as pl
from jax.experimental.pallas import tpu as pltpu
from jax.experimental.pallas import tpu_sc as plsc

M_CENTROIDS = 512
RADIUS = 0.15
KNBR = 32
EPS_BN = 1e-5
_NB = 8
_NN = 2048
_L = 16
_NW = 32
_WQ = _NW // _NB
_MW = M_CENTROIDS // _WQ
_GCHUNK = 256


def _prep_body(ptsT_ref, featT_ref, w1xT_ref, w1fT_ref, h_ref):
    ptsT = ptsT_ref[0]
    featT = featT_ref[0]
    h = jnp.dot(featT, w1fT_ref[...], preferred_element_type=jnp.float32)
    h = h + jnp.dot(ptsT, w1xT_ref[...], preferred_element_type=jnp.float32)
    h_ref[0] = h


def _prep_h(ptsT, featT, w1xT, w1fT):
    B, N, _ = ptsT.shape
    return pl.pallas_call(
        _prep_body,
        grid=(B,),
        in_specs=[
            pl.BlockSpec((1, N, 3), lambda b: (b, 0, 0)),
            pl.BlockSpec((1, N, featT.shape[2]), lambda b: (b, 0, 0)),
            pl.BlockSpec(w1xT.shape, lambda b: (0, 0)),
            pl.BlockSpec(w1fT.shape, lambda b: (0, 0)),
        ],
        out_specs=pl.BlockSpec((1, N, 128), lambda b: (b, 0, 0)),
        out_shape=jax.ShapeDtypeStruct((B, N, 128), jnp.float32),
    )(ptsT, featT, w1xT, w1fT)


def _mlp_body(g_ref, nxT_ref, w1xT_ref, w2T_ref, w3T_ref,
              s1_ref, b1_ref, s2_ref, b2_ref, s3_ref, b3_ref, out_ref):
    Mt = g_ref.shape[1]
    g = g_ref[0]
    nx = nxT_ref[0]
    cc = jnp.dot(nx, w1xT_ref[...], preferred_element_type=jnp.float32)
    y1 = g - cc[:, None, :]
    y1 = jnp.maximum(y1 * s1_ref[0][None, None, :] + b1_ref[0][None, None, :], 0.0)
    x1 = y1.reshape(Mt * KNBR, 128)
    y2 = jnp.dot(x1, w2T_ref[...], preferred_element_type=jnp.float32)
    y2 = jnp.maximum(y2 * s2_ref[0][None, :] + b2_ref[0][None, :], 0.0)
    y3 = jnp.dot(y2, w3T_ref[...], preferred_element_type=jnp.float32)
    y3 = jnp.maximum(y3 * s3_ref[0][None, :] + b3_ref[0][None, :], 0.0)
    m3 = y3.reshape(Mt, KNBR, 256)
    r = m3[:, 0, :]
    for k in range(1, KNBR):
        r = jnp.maximum(r, m3[:, k, :])
    out_ref[0] = r


def _mlp_maxpool(G, new_ptsT, w1xT, w2T, w3T, s1, b1, s2, b2, s3, b3):
    B, M, K, _ = G.shape
    Mt = 128
    vec = lambda v: v.reshape(1, -1)
    return pl.pallas_call(
        _mlp_body,
        grid=(B, M // Mt),
        in_specs=[
            pl.BlockSpec((1, Mt, K, 128), lambda b, m: (b, m, 0, 0)),
            pl.BlockSpec((1, Mt, 3), lambda b, m: (b, m, 0)),
            pl.BlockSpec((3, 128), lambda b, m: (0, 0)),
            pl.BlockSpec((128, 128), lambda b, m: (0, 0)),
            pl.BlockSpec((128, 256), lambda b, m: (0, 0)),
            pl.BlockSpec((1, 128), lambda b, m: (0, 0)),
            pl.BlockSpec((1, 128), lambda b, m: (0, 0)),
            pl.BlockSpec((1, 128), lambda b, m: (0, 0)),
            pl.BlockSpec((1, 128), lambda b, m: (0, 0)),
            pl.BlockSpec((1, 256), lambda b, m: (0, 0)),
            pl.BlockSpec((1, 256), lambda b, m: (0, 0)),
        ],
        out_specs=pl.BlockSpec((1, Mt, 256), lambda b, m: (b, m, 0)),
        out_shape=jax.ShapeDtypeStruct((B, M, 256), jnp.float32),
    )(G, new_ptsT, w1xT, w2T, w3T,
      vec(s1), vec(b1), vec(s2), vec(b2), vec(s3), vec(b3))


def _sc_sparse_body(xyz_hbm, h_hbm, npts_hbm, g_hbm,
                    x_v, y_v, z_v, mind_v, fps_v, bidx_v, npts_v, rows_v,
                    rows2_v, sem, sem2):
    cix = lax.axis_index("c")
    six = lax.axis_index("s")
    wid = six * 2 + cix
    b = wid // _WQ
    q = wid % _WQ

    iota = lax.iota(jnp.int32, _L)
    zeros16 = jnp.full((_L,), b * 0, jnp.int32)
    lane0 = iota == 0

    pltpu.sync_copy(xyz_hbm.at[pl.ds(b * 3 * _NN, _NN)], x_v)
    pltpu.sync_copy(xyz_hbm.at[pl.ds(b * 3 * _NN + _NN, _NN)], y_v)
    pltpu.sync_copy(xyz_hbm.at[pl.ds(b * 3 * _NN + 2 * _NN, _NN)], z_v)

    big = jnp.full((_L,), 1e10, jnp.float32)

    def init_body(j, carry):
        mind_v[pl.ds(j * _L, _L)] = big
        return carry
    lax.fori_loop(0, _NN // _L, init_body, 0)

    def initf_body(j, carry):
        fps_v[pl.ds(j * _L, _L)] = iota * 0
        return carry
    lax.fori_loop(0, M_CENTROIDS // _L, initf_body, 0)
    def initb_body(j, carry):
        bidx_v[pl.ds(j * _L, _L)] = iota * 0 + b * _NN
        return carry
    lax.fori_loop(0, _MW * KNBR // _L, initb_body, 0)
    xv0 = x_v[pl.ds(0, _L)]
    yv0 = y_v[pl.ds(0, _L)]
    zv0 = z_v[pl.ds(0, _L)]
    nbig = jnp.float32(-3.4e38)
    lastx = jnp.full((_L,), jnp.max(jnp.where(lane0, xv0, nbig)), jnp.float32)
    lasty = jnp.full((_L,), jnp.max(jnp.where(lane0, yv0, nbig)), jnp.float32)
    lastz = jnp.full((_L,), jnp.max(jnp.where(lane0, zv0, nbig)), jnp.float32)
    neginf = jnp.full((_L,), -3.4e38, jnp.float32)

    def fps_step(i, carry):
        lx, ly, lz = carry

        def sweep(jj, c2):
            bestd, besti = c2
            for u in range(4):
                off = jj * (4 * _L) + u * _L
                xv = x_v[pl.ds(off, _L)]
                yv = y_v[pl.ds(off, _L)]
                zv = z_v[pl.ds(off, _L)]
                dx = xv - lx
                dy = yv - ly
                dz = zv - lz
                d2 = (dx * dx + dy * dy) + dz * dz
                md = jnp.minimum(mind_v[pl.ds(off, _L)], d2)
                mind_v[pl.ds(off, _L)] = md
                upd = md > bestd
                bestd = jnp.where(upd, md, bestd)
                besti = jnp.where(upd, iota + off, besti)
            return bestd, besti

        bestd, besti = lax.fori_loop(0, _NN // (4 * _L), sweep,
                                     (neginf, zeros16))
        mx = jnp.max(bestd)
        cand = jnp.where(bestd == mx, besti, _NN)
        nxt = jnp.min(cand)
        nxtv = jnp.full((_L,), nxt, jnp.int32)
        plsc.store_scatter(fps_v, [jnp.full((_L,), i, jnp.int32)], nxtv,
                           mask=lane0)
        return (plsc.load_gather(x_v, [nxtv]),
                plsc.load_gather(y_v, [nxtv]),
                plsc.load_gather(z_v, [nxtv]))

    lax.fori_loop(1, M_CENTROIDS, fps_step, (lastx, lasty, lastz))

    for g in range(_MW // _L):
        cidxv = fps_v[pl.ds(q * _MW + g * _L, _L)]
        cx = plsc.load_gather(x_v, [cidxv])
        cy = plsc.load_gather(y_v, [cidxv])
        cz = plsc.load_gather(z_v, [cidxv])
        rowbase = (g * _L + iota) * 3
        plsc.store_scatter(npts_v, [rowbase], cx)
        plsc.store_scatter(npts_v, [rowbase + 1], cy)
        plsc.store_scatter(npts_v, [rowbase + 2], cz)
    pltpu.sync_copy(
        npts_v,
        npts_hbm.at[pl.ds(b * M_CENTROIDS * 3 + q * _MW * 3, _MW * 3)])

    r2 = jnp.float32(RADIUS * RADIUS)

    def ball_pair(cp, carry):
        cm_a = cp * 2
        pos_all = q * _MW + cm_a
        vbase = (pos_all // _L) * _L
        lane = pos_all % _L
        grp = fps_v[pl.ds(vbase, _L)]
        cs_a = jnp.max(jnp.where(iota == lane, grp, -1))
        cs_b = jnp.max(jnp.where(iota == lane + 1, grp, -1))
        csv_a = jnp.full((_L,), cs_a, jnp.int32)
        csv_b = jnp.full((_L,), cs_b, jnp.int32)
        cxa = plsc.load_gather(x_v, [csv_a])
        cya = plsc.load_gather(y_v, [csv_a])
        cza = plsc.load_gather(z_v, [csv_a])
        cxb = plsc.load_gather(x_v, [csv_b])
        cyb = plsc.load_gather(y_v, [csv_b])
        czb = plsc.load_gather(z_v, [csv_b])
        base_a = cm_a * KNBR
        base_bb = base_a + KNBR

        def scan(j, carry2):
            cnt_a, cnt_b = carry2
            ms = []
            for u in range(4):
                off = j * (4 * _L) + u * _L
                xv = x_v[pl.ds(off, _L)]
                yv = y_v[pl.ds(off, _L)]
                zv = z_v[pl.ds(off, _L)]
                dxa = xv - cxa
                dya = yv - cya
                dza = zv - cza
                d2a = (dxa * dxa + dya * dya) + dza * dza
                dxb = xv - cxb
                dyb = yv - cyb
                dzb = zv - czb
                d2b = (dxb * dxb + dyb * dyb) + dzb * dzb
                mska = d2a < r2
                mskb = d2b < r2
                ms.append((
                    mska, plsc.cumsum(mska.astype(jnp.int32)),
                    plsc.all_reduce_population_count(mska),
                    mskb, plsc.cumsum(mskb.astype(jnp.int32)),
                    plsc.all_reduce_population_count(mskb)))
            for u in range(4):
                off = j * (4 * _L) + u * _L
                mska, cuma, pca, mskb, cumb, pcb = ms[u]
                gidx = iota + (off + b * _NN)
                pos_a = cnt_a + cuma - 1
                wma = mska & (pos_a < KNBR)
                plsc.store_scatter(bidx_v, [base_a + pos_a], gidx, mask=wma)
                cnt_a = cnt_a + pca
                pos_b = cnt_b + cumb - 1
                wmb = mskb & (pos_b < KNBR)
                plsc.store_scatter(bidx_v, [base_bb + pos_b], gidx, mask=wmb)
                cnt_b = cnt_b + pcb
            return cnt_a, cnt_b

        cnt_a, cnt_b = lax.fori_loop(0, _NN // (4 * _L), scan,
                                     (zeros16, zeros16))

        for base_c, cnt in ((base_a, cnt_a), (base_bb, cnt_b)):
            firstv = plsc.load_gather(
                bidx_v, [jnp.full((_L,), base_c, jnp.int32)])
            for u in range(KNBR // _L):
                sl = pl.ds(base_c + u * _L, _L)
                cur = bidx_v[sl]
                lanepos = iota + u * _L
                bidx_v[sl] = jnp.where(lanepos < cnt, cur, firstv)
        return carry

    lax.fori_loop(0, _MW // 2, ball_pair, 0)

    nch = _MW * KNBR // _GCHUNK
    rbase = (b * M_CENTROIDS + q * _MW) * KNBR

    def _gather_chunk(i, buf, s):
        idx_sl = bidx_v.at[pl.ds(i * _GCHUNK, _GCHUNK)]
        return pltpu.async_copy(h_hbm.at[idx_sl], buf, s)

    cps = [None, None]
    cps[0] = _gather_chunk(0, rows_v, sem)
    for i in range(nch):
        nxt_i = i + 1
        if nxt_i < nch:
            cps[nxt_i % 2] = _gather_chunk(
                nxt_i, rows_v if nxt_i % 2 == 0 else rows2_v,
                sem if nxt_i % 2 == 0 else sem2)
        cps[i % 2].wait()
        buf = rows_v if i % 2 == 0 else rows2_v
        pltpu.sync_copy(buf, g_hbm.at[pl.ds(rbase + i * _GCHUNK, _GCHUNK), :])


def _sc_sparse(xyz, H):
    B, _, N = xyz.shape
    mesh = plsc.VectorSubcoreMesh(core_axis_name="c", subcore_axis_name="s")
    f = pl.kernel(
        _sc_sparse_body,
        mesh=mesh,
        compiler_params=pltpu.CompilerParams(needs_layout_passes=False),
        out_type=[
            jax.ShapeDtypeStruct((B * M_CENTROIDS * 3,), jnp.float32),
            jax.ShapeDtypeStruct((B * M_CENTROIDS * KNBR, 128), jnp.float32),
        ],
        scratch_types=[
            pltpu.VMEM((N,), jnp.float32),
            pltpu.VMEM((N,), jnp.float32),
            pltpu.VMEM((N,), jnp.float32),
            pltpu.VMEM((N,), jnp.float32),
            pltpu.VMEM((M_CENTROIDS,), jnp.int32),
            pltpu.VMEM((_MW * KNBR,), jnp.int32),
            pltpu.VMEM((_MW * 3,), jnp.float32),
            pltpu.VMEM((_GCHUNK, 128), jnp.float32),
            pltpu.VMEM((_GCHUNK, 128), jnp.float32),
            pltpu.SemaphoreType.DMA,
            pltpu.SemaphoreType.DMA,
        ],
    )
    return f(xyz.reshape(-1), H.reshape(-1, 128))


def _gather_rows(x, idx):
    return jax.vmap(lambda xb, ib: xb[ib])(x, idx)


def _fps_scaffold(pts, M):
    B, N, _ = pts.shape
    def body(i, state):
        idxs, min_d, last = state
        last_pt = _gather_rows(pts, last)
        d = jnp.sum((pts - last_pt[:, None, :]) ** 2, axis=-1)
        min_d = jnp.minimum(min_d, d)
        nxt = jnp.argmax(min_d, axis=-1).astype(jnp.int32)
        idxs = idxs.at[:, i].set(nxt)
        return (idxs, min_d, nxt)
    state0 = (jnp.zeros((B, M), jnp.int32),
              jnp.full((B, N), 1e10, jnp.float32),
              jnp.zeros((B,), jnp.int32))
    idxs, _, _ = lax.fori_loop(1, M, body, state0)
    return idxs


def _ball_scaffold(new_pts, pts, radius, K):
    d2 = jnp.sum((new_pts[:, :, None, :] - pts[:, None, :, :]) ** 2, axis=-1)
    N = pts.shape[1]
    cand = jnp.where(d2 < radius * radius,
                     jnp.arange(N, dtype=jnp.int32)[None, None, :], N)
    idx_sorted = jnp.sort(cand, axis=-1)[:, :, :K]
    first = idx_sorted[:, :, 0:1]
    idx = jnp.where(idx_sorted < N, idx_sorted, first)
    return jnp.minimum(idx, N - 1).astype(jnp.int32)


def kernel(xyz, feature, W1, gamma1, beta1, W2, gamma2, beta2, W3, gamma3, beta3):
    B, _, N = xyz.shape
    inv = 1.0 / jnp.sqrt(1.0 + EPS_BN)
    s1, s2, s3 = gamma1 * inv, gamma2 * inv, gamma3 * inv

    ptsT = jnp.transpose(xyz, (0, 2, 1))
    featT = jnp.transpose(feature, (0, 2, 1))
    w1xT = jnp.transpose(W1[:, :3])
    w1fT = jnp.transpose(W1[:, 3:])
    w2T = jnp.transpose(W2)
    w3T = jnp.transpose(W3)

    H = _prep_h(ptsT, featT, w1xT, w1fT)

    npts_flat, G_flat = _sc_sparse(xyz, H)
    new_pts = npts_flat.reshape(B, M_CENTROIDS, 3)
    G = G_flat.reshape(B, M_CENTROIDS, KNBR, 128)

    nf = _mlp_maxpool(G, new_pts, w1xT, w2T, w3T, s1, beta1, s2, beta2, s3, beta3)
    new_xyz = jnp.transpose(new_pts, (0, 2, 1))
    new_feature = jnp.transpose(nf, (0, 2, 1))
    return (new_xyz, new_feature)


# speedup vs baseline: 2.2245x; 1.0014x over previous
import functools
import jax
import jax.numpy as jnp
from jax import lax
from jax.experimental import pallas as pl
from jax.experimental.pallas import tpu as pltpu
from jax.experimental.pallas import tpu_sc as plsc

M_CENTROIDS = 512
RADIUS = 0.15
KNBR = 32
EPS_BN = 1e-5
_NB = 8
_NN = 2048
_L = 16
_NW = 32
_WQ = _NW // _NB
_MW = M_CENTROIDS // _WQ
_GCHUNK = 256


def _prep_body(ptsT_ref, featT_ref, w1xT_ref, w1fT_ref, h_ref):
    ptsT = ptsT_ref[0]
    featT = featT_ref[0]
    h = jnp.dot(featT, w1fT_ref[...], preferred_element_type=jnp.float32)
    h = h + jnp.dot(ptsT, w1xT_ref[...], preferred_element_type=jnp.float32)
    h_ref[0] = h


def _prep_h(ptsT, featT, w1xT, w1fT):
    B, N, _ = ptsT.shape
    return pl.pallas_call(
        _prep_body,
        grid=(B,),
        in_specs=[
            pl.BlockSpec((1, N, 3), lambda b: (b, 0, 0)),
            pl.BlockSpec((1, N, featT.shape[2]), lambda b: (b, 0, 0)),
            pl.BlockSpec(w1xT.shape, lambda b: (0, 0)),
            pl.BlockSpec(w1fT.shape, lambda b: (0, 0)),
        ],
        out_specs=pl.BlockSpec((1, N, 128), lambda b: (b, 0, 0)),
        out_shape=jax.ShapeDtypeStruct((B, N, 128), jnp.float32),
    )(ptsT, featT, w1xT, w1fT)


def _mlp_body(g_ref, nxT_ref, w1xT_ref, w2T_ref, w3T_ref,
              s1_ref, b1_ref, s2_ref, b2_ref, s3_ref, b3_ref, out_ref):
    Mt = g_ref.shape[1]
    g = g_ref[0]
    nx = nxT_ref[0]
    cc = jnp.dot(nx, w1xT_ref[...], preferred_element_type=jnp.float32)
    y1 = g - cc[:, None, :]
    y1 = jnp.maximum(y1 * s1_ref[0][None, None, :] + b1_ref[0][None, None, :], 0.0)
    x1 = y1.reshape(Mt * KNBR, 128)
    y2 = jnp.dot(x1, w2T_ref[...], preferred_element_type=jnp.float32)
    y2 = jnp.maximum(y2 * s2_ref[0][None, :] + b2_ref[0][None, :], 0.0)
    y3 = jnp.dot(y2, w3T_ref[...], preferred_element_type=jnp.float32)
    y3 = jnp.maximum(y3 * s3_ref[0][None, :] + b3_ref[0][None, :], 0.0)
    m3 = y3.reshape(Mt, KNBR, 256)
    r = m3[:, 0, :]
    for k in range(1, KNBR):
        r = jnp.maximum(r, m3[:, k, :])
    out_ref[0] = r


def _mlp_maxpool(G, new_ptsT, w1xT, w2T, w3T, s1, b1, s2, b2, s3, b3):
    B, M, K, _ = G.shape
    Mt = 128
    vec = lambda v: v.reshape(1, -1)
    return pl.pallas_call(
        _mlp_body,
        grid=(B, M // Mt),
        in_specs=[
            pl.BlockSpec((1, Mt, K, 128), lambda b, m: (b, m, 0, 0)),
            pl.BlockSpec((1, Mt, 3), lambda b, m: (b, m, 0)),
            pl.BlockSpec((3, 128), lambda b, m: (0, 0)),
            pl.BlockSpec((128, 128), lambda b, m: (0, 0)),
            pl.BlockSpec((128, 256), lambda b, m: (0, 0)),
            pl.BlockSpec((1, 128), lambda b, m: (0, 0)),
            pl.BlockSpec((1, 128), lambda b, m: (0, 0)),
            pl.BlockSpec((1, 128), lambda b, m: (0, 0)),
            pl.BlockSpec((1, 128), lambda b, m: (0, 0)),
            pl.BlockSpec((1, 256), lambda b, m: (0, 0)),
            pl.BlockSpec((1, 256), lambda b, m: (0, 0)),
        ],
        out_specs=pl.BlockSpec((1, Mt, 256), lambda b, m: (b, m, 0)),
        out_shape=jax.ShapeDtypeStruct((B, M, 256), jnp.float32),
    )(G, new_ptsT, w1xT, w2T, w3T,
      vec(s1), vec(b1), vec(s2), vec(b2), vec(s3), vec(b3))


def _sc_sparse_body(xyz_hbm, h_hbm, npts_hbm, g_hbm,
                    x_v, y_v, z_v, mind_v, fps_v, bidx_v, npts_v, rows_v,
                    rows2_v, sem, sem2):
    cix = lax.axis_index("c")
    six = lax.axis_index("s")
    wid = six * 2 + cix
    b = wid // _WQ
    q = wid % _WQ

    iota = lax.iota(jnp.int32, _L)
    zeros16 = jnp.full((_L,), b * 0, jnp.int32)
    lane0 = iota == 0

    pltpu.sync_copy(xyz_hbm.at[pl.ds(b * 3 * _NN, _NN)], x_v)
    pltpu.sync_copy(xyz_hbm.at[pl.ds(b * 3 * _NN + _NN, _NN)], y_v)
    pltpu.sync_copy(xyz_hbm.at[pl.ds(b * 3 * _NN + 2 * _NN, _NN)], z_v)

    big = jnp.full((_L,), 1e10, jnp.float32)

    def init_body(j, carry):
        mind_v[pl.ds(j * _L, _L)] = big
        return carry
    lax.fori_loop(0, _NN // _L, init_body, 0)

    def initf_body(j, carry):
        fps_v[pl.ds(j * _L, _L)] = iota * 0
        return carry
    lax.fori_loop(0, M_CENTROIDS // _L, initf_body, 0)
    def initb_body(j, carry):
        bidx_v[pl.ds(j * _L, _L)] = iota * 0 + b * _NN
        return carry
    lax.fori_loop(0, _MW * KNBR // _L, initb_body, 0)
    xv0 = x_v[pl.ds(0, _L)]
    yv0 = y_v[pl.ds(0, _L)]
    zv0 = z_v[pl.ds(0, _L)]
    nbig = jnp.float32(-3.4e38)
    lastx = jnp.full((_L,), jnp.max(jnp.where(lane0, xv0, nbig)), jnp.float32)
    lasty = jnp.full((_L,), jnp.max(jnp.where(lane0, yv0, nbig)), jnp.float32)
    lastz = jnp.full((_L,), jnp.max(jnp.where(lane0, zv0, nbig)), jnp.float32)
    neginf = jnp.full((_L,), -3.4e38, jnp.float32)

    def fps_step(i, carry):
        lx, ly, lz = carry

        def sweep(jj, c2):
            bestd, besti = c2
            for u in range(4):
                off = jj * (4 * _L) + u * _L
                xv = x_v[pl.ds(off, _L)]
                yv = y_v[pl.ds(off, _L)]
                zv = z_v[pl.ds(off, _L)]
                dx = xv - lx
                dy = yv - ly
                dz = zv - lz
                d2 = (dx * dx + dy * dy) + dz * dz
                md = jnp.minimum(mind_v[pl.ds(off, _L)], d2)
                mind_v[pl.ds(off, _L)] = md
                upd = md > bestd
                bestd = jnp.where(upd, md, bestd)
                besti = jnp.where(upd, iota + off, besti)
            return bestd, besti

        bestd, besti = lax.fori_loop(0, _NN // (4 * _L), sweep,
                                     (neginf, zeros16))
        mx = jnp.max(bestd)
        cand = jnp.where(bestd == mx, besti, _NN)
        nxt = jnp.min(cand)
        nxtv = jnp.full((_L,), nxt, jnp.int32)
        plsc.store_scatter(fps_v, [jnp.full((_L,), i, jnp.int32)], nxtv,
                           mask=lane0)
        return (plsc.load_gather(x_v, [nxtv]),
                plsc.load_gather(y_v, [nxtv]),
                plsc.load_gather(z_v, [nxtv]))

    lax.fori_loop(1, M_CENTROIDS, fps_step, (lastx, lasty, lastz))

    for g in range(_MW // _L):
        cidxv = fps_v[pl.ds(q * _MW + g * _L, _L)]
        cx = plsc.load_gather(x_v, [cidxv])
        cy = plsc.load_gather(y_v, [cidxv])
        cz = plsc.load_gather(z_v, [cidxv])
        rowbase = (g * _L + iota) * 3
        plsc.store_scatter(npts_v, [rowbase], cx)
        plsc.store_scatter(npts_v, [rowbase + 1], cy)
        plsc.store_scatter(npts_v, [rowbase + 2], cz)
    pltpu.sync_copy(
        npts_v,
        npts_hbm.at[pl.ds(b * M_CENTROIDS * 3 + q * _MW * 3, _MW * 3)])

    r2 = jnp.float32(RADIUS * RADIUS)

    def ball_pair(cp, carry):
        cm_a = cp * 2
        pos_all = q * _MW + cm_a
        vbase = (pos_all // _L) * _L
        lane = pos_all % _L
        grp = fps_v[pl.ds(vbase, _L)]
        cs_a = jnp.max(jnp.where(iota == lane, grp, -1))
        cs_b = jnp.max(jnp.where(iota == lane + 1, grp, -1))
        csv_a = jnp.full((_L,), cs_a, jnp.int32)
        csv_b = jnp.full((_L,), cs_b, jnp.int32)
        cxa = plsc.load_gather(x_v, [csv_a])
        cya = plsc.load_gather(y_v, [csv_a])
        cza = plsc.load_gather(z_v, [csv_a])
        cxb = plsc.load_gather(x_v, [csv_b])
        cyb = plsc.load_gather(y_v, [csv_b])
        czb = plsc.load_gather(z_v, [csv_b])
        base_a = cm_a * KNBR
        base_bb = base_a + KNBR

        def scan(j, carry2):
            cnt_a, cnt_b = carry2
            ms = []
            for u in range(4):
                off = j * (4 * _L) + u * _L
                xv = x_v[pl.ds(off, _L)]
                yv = y_v[pl.ds(off, _L)]
                zv = z_v[pl.ds(off, _L)]
                dxa = xv - cxa
                dya = yv - cya
                dza = zv - cza
                d2a = (dxa * dxa + dya * dya) + dza * dza
                dxb = xv - cxb
                dyb = yv - cyb
                dzb = zv - czb
                d2b = (dxb * dxb + dyb * dyb) + dzb * dzb
                mska = d2a < r2
                mskb = d2b < r2
                ms.append((
                    mska, plsc.cumsum(mska.astype(jnp.int32)),
                    plsc.all_reduce_population_count(mska),
                    mskb, plsc.cumsum(mskb.astype(jnp.int32)),
                    plsc.all_reduce_population_count(mskb)))
            for u in range(4):
                off = j * (4 * _L) + u * _L
                mska, cuma, pca, mskb, cumb, pcb = ms[u]
                gidx = iota + (off + b * _NN)
                pos_a = cnt_a + cuma - 1
                wma = mska & (pos_a < KNBR)
                plsc.store_scatter(bidx_v, [base_a + pos_a], gidx, mask=wma)
                cnt_a = cnt_a + pca
                pos_b = cnt_b + cumb - 1
                wmb = mskb & (pos_b < KNBR)
                plsc.store_scatter(bidx_v, [base_bb + pos_b], gidx, mask=wmb)
                cnt_b = cnt_b + pcb
            return cnt_a, cnt_b

        cnt_a, cnt_b = lax.fori_loop(0, _NN // (4 * _L), scan,
                                     (zeros16, zeros16))

        for base_c, cnt in ((base_a, cnt_a), (base_bb, cnt_b)):
            firstv = plsc.load_gather(
                bidx_v, [jnp.full((_L,), base_c, jnp.int32)])
            for u in range(KNBR // _L):
                sl = pl.ds(base_c + u * _L, _L)
                cur = bidx_v[sl]
                lanepos = iota + u * _L
                bidx_v[sl] = jnp.where(lanepos < cnt, cur, firstv)
        return carry

    lax.fori_loop(0, _MW // 2, ball_pair, 0)

    nch = _MW * KNBR // _GCHUNK
    rbase = (b * M_CENTROIDS + q * _MW) * KNBR

    def _gather_chunk(i, buf, s):
        idx_sl = bidx_v.at[pl.ds(i * _GCHUNK, _GCHUNK)]
        return pltpu.async_copy(h_hbm.at[idx_sl], buf, s)

    cps = [None, None]
    cps[0] = _gather_chunk(0, rows_v, sem)
    for i in range(nch):
        nxt_i = i + 1
        if nxt_i < nch:
            cps[nxt_i % 2] = _gather_chunk(
                nxt_i, rows_v if nxt_i % 2 == 0 else rows2_v,
                sem if nxt_i % 2 == 0 else sem2)
        cps[i % 2].wait()
        buf = rows_v if i % 2 == 0 else rows2_v
        pltpu.sync_copy(buf, g_hbm.at[pl.ds(rbase + i * _GCHUNK, _GCHUNK), :])


def _sc_sparse(xyz, H):
    B, _, N = xyz.shape
    mesh = plsc.VectorSubcoreMesh(core_axis_name="c", subcore_axis_name="s")
    f = pl.kernel(
        _sc_sparse_body,
        mesh=mesh,
        compiler_params=pltpu.CompilerParams(needs_layout_passes=False),
        out_type=[
            jax.ShapeDtypeStruct((B * M_CENTROIDS * 3,), jnp.float32),
            jax.ShapeDtypeStruct((B * M_CENTROIDS * KNBR, 128), jnp.float32),
        ],
        scratch_types=[
            pltpu.VMEM((N,), jnp.float32),
            pltpu.VMEM((N,), jnp.float32),
            pltpu.VMEM((N,), jnp.float32),
            pltpu.VMEM((N,), jnp.float32),
            pltpu.VMEM((M_CENTROIDS,), jnp.int32),
            pltpu.VMEM((_MW * KNBR,), jnp.int32),
            pltpu.VMEM((_MW * 3,), jnp.float32),
            pltpu.VMEM((_GCHUNK, 128), jnp.float32),
            pltpu.VMEM((_GCHUNK, 128), jnp.float32),
            pltpu.SemaphoreType.DMA,
            pltpu.SemaphoreType.DMA,
        ],
    )
    return f(xyz.reshape(-1), H.reshape(-1, 128))


def kernel(xyz, feature, W1, gamma1, beta1, W2, gamma2, beta2, W3, gamma3, beta3):
    B, _, N = xyz.shape
    inv = 1.0 / jnp.sqrt(1.0 + EPS_BN)
    s1, s2, s3 = gamma1 * inv, gamma2 * inv, gamma3 * inv

    ptsT = jnp.transpose(xyz, (0, 2, 1))
    featT = jnp.transpose(feature, (0, 2, 1))
    w1xT = jnp.transpose(W1[:, :3])
    w1fT = jnp.transpose(W1[:, 3:])
    w2T = jnp.transpose(W2)
    w3T = jnp.transpose(W3)

    H = _prep_h(ptsT, featT, w1xT, w1fT)

    npts_flat, G_flat = _sc_sparse(xyz, H)
    new_pts = npts_flat.reshape(B, M_CENTROIDS, 3)
    G = G_flat.reshape(B, M_CENTROIDS, KNBR, 128)

    nf = _mlp_maxpool(G, new_pts, w1xT, w2T, w3T, s1, beta1, s2, beta2, s3, beta3)
    new_xyz = jnp.transpose(new_pts, (0, 2, 1))
    new_feature = jnp.transpose(nf, (0, 2, 1))
    return (new_xyz, new_feature)
